# baseline (speedup 1.0000x reference)
import functools

import jax
import jax.numpy as jnp
from jax import lax
from jax.experimental import pallas as pl
from jax.experimental.pallas import tpu as pltpu
from jax.experimental.pallas import tpu_sc as plsc

HH = 256
E_PAD = 53248
A_PAD = 5120
O_PAD = 4096
G_PAD = 1024
NW = 32
SENTINEL = 1 << 20

f32 = jnp.float32
i32 = jnp.int32


def _padr(x, n, val=0.0):
    pads = ((0, n - x.shape[0]),) + ((0, 0),) * (x.ndim - 1)
    return jnp.pad(x, pads, constant_values=val)


def _sc_mesh():
    return plsc.VectorSubcoreMesh(core_axis_name="c", subcore_axis_name="s")


_SC_PARAMS = pltpu.CompilerParams(needs_layout_passes=False)


@functools.partial(jax.jit, static_argnames=())
def _sc_gather(table, idx):
    per_w = E_PAD // NW
    C = 128
    nch = per_w // C

    @functools.partial(
        pl.kernel,
        mesh=_sc_mesh(),
        out_type=jax.ShapeDtypeStruct((E_PAD, HH), f32),
        compiler_params=_SC_PARAMS,
        scratch_types=[
            pltpu.VMEM((C,), i32),
            pltpu.VMEM((C, HH), f32),
            pltpu.VMEM((C, HH), f32),
            pltpu.SemaphoreType.DMA,
            pltpu.SemaphoreType.DMA,
        ],
    )
    def k(table_hbm, idx_hbm, out_hbm, idx_v, rows_a, rows_b, sem_a, sem_b):
        wid = lax.axis_index("s") * 2 + lax.axis_index("c")
        base0 = wid * per_w

        def body(kk, _):
            base = base0 + kk * C
            pltpu.sync_copy(idx_hbm.at[pl.ds(base, C)], idx_v)
            pltpu.async_copy(table_hbm.at[idx_v], rows_a, sem_a).wait()
            pltpu.sync_copy(rows_a, out_hbm.at[pl.ds(base, C)])
            return 0

        lax.fori_loop(0, nch, body, 0, unroll=False)

    return k(table, idx)


def _sc_segmax(m, dst):
    rows = A_PAD // NW
    DCH = 2048
    nch = E_PAD // DCH
    G = 64

    @functools.partial(
        pl.kernel,
        mesh=_sc_mesh(),
        out_type=jax.ShapeDtypeStruct((A_PAD, HH), f32),
        compiler_params=_SC_PARAMS,
        scratch_types=[
            pltpu.VMEM((rows, HH), f32),
            pltpu.VMEM((DCH,), i32),
            pltpu.VMEM((E_PAD + 16,), i32),
            pltpu.VMEM((G,), i32),
            pltpu.VMEM((G, HH), f32),
            pltpu.SemaphoreType.DMA,
        ],
    )
    def k(m_hbm, dst_hbm, agg_hbm, acc_v, dbuf_v, hits_v, idxt_v, rows_v, sem):
        wid = lax.axis_index("s") * 2 + lax.axis_index("c")
        lo = wid * rows
        hi = lo + rows
        neginf = jnp.full((16,), -jnp.inf, f32)

        def init_row(r, _):
            for c in range(HH // 16):
                acc_v[r, pl.ds(c * 16, 16)] = neginf
            return 0

        lax.fori_loop(0, rows, init_row, 0, unroll=False)

        lane = lax.broadcasted_iota(i32, (16,), 0)

        def chunk_body(kk, cnt):
            pltpu.sync_copy(dst_hbm.at[pl.ds(kk * DCH, DCH)], dbuf_v)

            def vbody(v, cnt):
                d = dbuf_v[pl.ds(v * 16, 16)]
                msk = (d >= lo) & (d < hi)
                eid = kk * DCH + v * 16 + lane
                packed = ((d - lo) << 16) | eid
                pos = plsc.cumsum(msk.astype(i32))
                plsc.store_scatter(hits_v, [cnt + pos - 1], packed, mask=msk)
                return cnt + pos[15]

            return lax.fori_loop(0, DCH // 16, vbody, cnt, unroll=False)

        cnt = lax.fori_loop(0, nch, chunk_body, 0, unroll=False)

        ng = (cnt + G - 1) // G

        def gbody(g, _):
            base = g * G
            for vv in range(G // 16):
                pos = base + vv * 16
                p = hits_v[pl.ds(pos, 16)]
                valid = (pos + lane) < cnt
                idxt_v[pl.ds(vv * 16, 16)] = jnp.where(valid, p & 0xFFFF, 0)
            pltpu.async_copy(m_hbm.at[idxt_v], rows_v, sem).wait()
            count = jnp.minimum(G, cnt - base)

            def rbody(r, _):
                pk = hits_v[pl.ds(base + r, 16)][0]
                off = pk >> 16

                @pl.when(r < count)
                def _():
                    for c in range(HH // 16):
                        sl = pl.ds(c * 16, 16)
                        acc_v[off, sl] = jnp.maximum(acc_v[off, sl], rows_v[r, sl])

                return 0

            lax.fori_loop(0, G, rbody, 0, unroll=False)
            return 0

        lax.fori_loop(0, ng, gbody, 0, unroll=False)

        pltpu.sync_copy(acc_v, agg_hbm.at[pl.ds(lo, rows)])

    return k(m, dst)


def _mm(x, w):
    def body(x_ref, w_ref, o_ref):
        o_ref[...] = jnp.dot(x_ref[...], w_ref[...], preferred_element_type=f32)

    return pl.pallas_call(
        body,
        out_shape=jax.ShapeDtypeStruct((x.shape[0], w.shape[1]), f32),
    )(x, w)


def _edge_init(attr, w1, b1, w2, b2):
    R = 2048
    grid = E_PAD // R

    def body(a_ref, w1_ref, b1_ref, w2_ref, b2_ref, o_ref):
        h = jnp.maximum(
            jnp.dot(a_ref[...], w1_ref[...], preferred_element_type=f32)
            + b1_ref[...], 0.0)
        o_ref[...] = jnp.dot(h, w2_ref[...], preferred_element_type=f32) + b2_ref[...]

    full = lambda s: pl.BlockSpec(s, lambda i: (0, 0))
    return pl.pallas_call(
        body,
        grid=(grid,),
        in_specs=[
            pl.BlockSpec((R, 16), lambda i: (i, 0)),
            full((16, HH)), full((1, HH)), full((HH, HH)), full((1, HH)),
        ],
        out_specs=pl.BlockSpec((R, HH), lambda i: (i, 0)),
        out_shape=jax.ShapeDtypeStruct((E_PAD, HH), f32),
    )(attr, w1, b1.reshape(1, HH), w2, b2.reshape(1, HH))


def _edge_mlp(ea, g0, g1, w1e, b1, w2, b2, wf1, bf1, wf2, bf2):
    R = 2048
    grid = E_PAD // R

    def body(ea_ref, g0_ref, g1_ref, w1e_ref, b1_ref, w2_ref, b2_ref,
             wf1_ref, bf1_ref, wf2_ref, bf2_ref, ean_ref, m_ref):
        a = ea_ref[...]
        pre = (jnp.dot(a, w1e_ref[...], preferred_element_type=f32)
               + g0_ref[...] + g1_ref[...] + b1_ref[...])
        h = jnp.maximum(pre, 0.0)
        ean = a + jnp.dot(h, w2_ref[...], preferred_element_type=f32) + b2_ref[...]
        ean_ref[...] = ean
        h2 = jnp.maximum(
            jnp.dot(ean, wf1_ref[...], preferred_element_type=f32) + bf1_ref[...], 0.0)
        m_ref[...] = jnp.dot(h2, wf2_ref[...], preferred_element_type=f32) + bf2_ref[...]

    eb = pl.BlockSpec((R, HH), lambda i: (i, 0))
    full = lambda s: pl.BlockSpec(s, lambda i: (0, 0))
    return pl.pallas_call(
        body,
        grid=(grid,),
        in_specs=[eb, eb, eb,
                  full((HH, HH)), full((1, HH)), full((HH, HH)), full((1, HH)),
                  full((HH, HH)), full((1, HH)), full((HH, HH)), full((1, HH))],
        out_specs=[eb, eb],
        out_shape=[jax.ShapeDtypeStruct((E_PAD, HH), f32),
                   jax.ShapeDtypeStruct((E_PAD, HH), f32)],
    )(ea, g0, g1, w1e, b1.reshape(1, HH), w2, b2.reshape(1, HH),
      wf1, bf1.reshape(1, HH), wf2, bf2.reshape(1, HH))


def _agent_update(xa, a0, a1, a2):
    def body(x_ref, a0_ref, a1_ref, a2_ref, o_ref):
        def fix(v):
            return jnp.where(jnp.isfinite(v), v, 0.0)
        o_ref[...] = x_ref[...] + jnp.maximum(
            jnp.maximum(fix(a0_ref[...]), fix(a1_ref[...])), fix(a2_ref[...]))

    return pl.pallas_call(
        body,
        out_shape=jax.ShapeDtypeStruct((A_PAD, HH), f32),
    )(xa, a0, a1, a2)


def _field(xa, act, w1v, w1a, b1, w2, b2):
    def body(x_ref, a_ref, w1v_ref, w1a_ref, b1_ref, w2_ref, b2_ref, o_ref):
        h = jnp.maximum(
            jnp.dot(x_ref[...], w1v_ref[...], preferred_element_type=f32)
            + jnp.dot(a_ref[...], w1a_ref[...], preferred_element_type=f32)
            + b1_ref[...], 0.0)
        o_ref[...] = jnp.dot(h, w2_ref[...], preferred_element_type=f32) + b2_ref[...]

    return pl.pallas_call(
        body,
        out_shape=jax.ShapeDtypeStruct((A_PAD, 1), f32),
    )(xa, act, w1v, w1a, b1.reshape(1, HH), w2, b2.reshape(1, 1))


def kernel(x_obstacle, x_agent, x_goal, action,
           edge_index_oa, edge_attr_oa,
           edge_index_aa, edge_attr_aa,
           edge_index_ga, edge_attr_ga, params):
    types = ["oa", "aa", "ga"]
    ei = {"oa": edge_index_oa, "aa": edge_index_aa, "ga": edge_index_ga}
    eattr = {"oa": edge_attr_oa, "aa": edge_attr_aa, "ga": edge_attr_ga}

    xo = _mm(_padr(x_obstacle, O_PAD), params["W_embed"])
    xa = _mm(_padr(x_agent, A_PAD), params["W_embed"])
    xg = _mm(_padr(x_goal, G_PAD), params["W_embed"])
    act = _padr(action, A_PAD)

    ea, idx0, idx1 = {}, {}, {}
    for t in types:
        pp = params["ee_" + t]
        ea[t] = _edge_init(_padr(eattr[t], E_PAD), pp["W1"], pp["b1"], pp["W2"], pp["b2"])
        e = ei[t]
        idx0[t] = jnp.pad(e[0], (0, E_PAD - e.shape[1]), constant_values=0).astype(i32)
        idx1[t] = jnp.pad(e[1], (0, E_PAD - e.shape[1]), constant_values=0).astype(i32)

    dsts = {t: jnp.pad(ei[t][1], (0, E_PAD - ei[t].shape[1]),
                       constant_values=SENTINEL).astype(i32) for t in types}

    for l in range(2):
        aggs = []
        for t in types:
            em = params["em_%d_%s" % (l, t)]
            W1 = em["W1"]
            W1e, W1s, W1d = W1[:HH], W1[HH:2 * HH], W1[2 * HH:]
            xs = xo if t == "oa" else (xa if t == "aa" else xg)
            p_src = _mm(xs, W1s)
            p_dst = _mm(xa, W1d)
            g0 = _sc_gather(p_src, idx0[t])
            g1 = _sc_gather(p_dst, idx1[t])
            fx = params["fx_%d_%s" % (l, t)]
            ea[t], m = _edge_mlp(ea[t], g0, g1, W1e, em["b1"], em["W2"], em["b2"],
                                 fx["W1"], fx["b1"], fx["W2"], fx["b2"])
            aggs.append(_sc_segmax(m, dsts[t]))
        xa = _agent_update(xa, aggs[0], aggs[1], aggs[2])

    fp = params["field"]
    W1f = fp["W1"]
    out = _field(xa, act, W1f[:HH], W1f[HH:], fp["b1"], fp["W2"], fp["b2"])
    return out[:x_agent.shape[0], 0]


# speedup vs baseline: 1.3153x; 1.3153x over previous
import functools

import jax
import jax.numpy as jnp
from jax import lax
from jax.experimental import pallas as pl
from jax.experimental.pallas import tpu as pltpu
from jax.experimental.pallas import tpu_sc as plsc

HH = 256
E_PAD = 53248
A_PAD = 5120
O_PAD = 4096
G_PAD = 1024
NW = 32
SENTINEL = 1 << 20

f32 = jnp.float32
i32 = jnp.int32


def _padr(x, n, val=0.0):
    pads = ((0, n - x.shape[0]),) + ((0, 0),) * (x.ndim - 1)
    return jnp.pad(x, pads, constant_values=val)


def _sc_mesh():
    return plsc.VectorSubcoreMesh(core_axis_name="c", subcore_axis_name="s")


_SC_PARAMS = pltpu.CompilerParams(needs_layout_passes=False)


def _sc_gather(table, idx):
    W = table.shape[1]
    per_w = E_PAD // NW
    C = 128
    nch = per_w // C
    NB = 3

    @functools.partial(
        pl.kernel,
        mesh=_sc_mesh(),
        out_type=jax.ShapeDtypeStruct((E_PAD, W), f32),
        compiler_params=_SC_PARAMS,
        scratch_types=[pltpu.VMEM((per_w,), i32)]
        + [pltpu.VMEM((C, W), f32)] * NB
        + [pltpu.SemaphoreType.DMA] * NB
        + [pltpu.SemaphoreType.DMA] * NB,
    )
    def k(table_hbm, idx_hbm, out_hbm, idx_v, *bufsems):
        rows = bufsems[:NB]
        gsem = bufsems[NB:2 * NB]
        wsem = bufsems[2 * NB:]
        wid = lax.axis_index("s") * 2 + lax.axis_index("c")
        base0 = wid * per_w
        pltpu.sync_copy(idx_hbm.at[pl.ds(base0, per_w)], idx_v)

        gathers = [None] * nch
        writes = [None] * nch
        for kk in range(nch):
            b = kk % NB
            if kk >= NB:
                writes[kk - NB].wait()
            gathers[kk] = pltpu.async_copy(
                table_hbm.at[idx_v.at[pl.ds(kk * C, C)]], rows[b], gsem[b])
            if kk >= 1:
                gathers[kk - 1].wait()
                writes[kk - 1] = pltpu.async_copy(
                    rows[(kk - 1) % NB],
                    out_hbm.at[pl.ds(base0 + (kk - 1) * C, C)],
                    wsem[(kk - 1) % NB])
        gathers[nch - 1].wait()
        writes[nch - 1] = pltpu.async_copy(
            rows[(nch - 1) % NB],
            out_hbm.at[pl.ds(base0 + (nch - 1) * C, C)],
            wsem[(nch - 1) % NB])
        for kk in range(nch - NB, nch):
            writes[kk].wait()

    return k(table, idx)


_ROWS = A_PAD // NW
_HCH = 4096


def _sc_hits(dst):
    DCH = 2048
    nch = E_PAD // DCH

    @functools.partial(
        pl.kernel,
        mesh=_sc_mesh(),
        out_type=[jax.ShapeDtypeStruct((NW, E_PAD), i32),
                  jax.ShapeDtypeStruct((NW, 16), i32)],
        compiler_params=_SC_PARAMS,
        scratch_types=[
            pltpu.VMEM((DCH,), i32),
            pltpu.VMEM((E_PAD + 16,), i32),
            pltpu.VMEM((16,), i32),
        ],
    )
    def k(dst_hbm, hits_hbm, cnt_hbm, dbuf_v, hits_v, cnt_v):
        wid = lax.axis_index("s") * 2 + lax.axis_index("c")
        lo = wid * _ROWS
        hi = lo + _ROWS
        lane = lax.broadcasted_iota(i32, (16,), 0)

        def chunk_body(kk, cnt):
            pltpu.sync_copy(dst_hbm.at[pl.ds(kk * DCH, DCH)], dbuf_v)

            def vbody(v, cnt):
                d = dbuf_v[pl.ds(v * 16, 16)]
                msk = (d >= lo) & (d < hi)
                eid = kk * DCH + v * 16 + lane
                packed = ((d - lo) << 16) | eid
                pos = plsc.cumsum(msk.astype(i32))
                plsc.store_scatter(hits_v, [cnt + pos - 1], packed, mask=msk)
                return cnt + pos[15]

            return lax.fori_loop(0, DCH // 16, vbody, cnt, unroll=False)

        cnt = lax.fori_loop(0, nch, chunk_body, 0, unroll=False)

        cnt_v[...] = jnp.zeros((16,), i32) + cnt
        pltpu.sync_copy(cnt_v, cnt_hbm.at[wid])
        for c in range(E_PAD // _HCH):
            @pl.when(c * _HCH < cnt)
            def _():
                pltpu.sync_copy(hits_v.at[pl.ds(c * _HCH, _HCH)],
                                hits_hbm.at[wid, pl.ds(c * _HCH, _HCH)])

    return k(dst)


def _sc_segmax(m, hits, cnt):
    G = 64

    @functools.partial(
        pl.kernel,
        mesh=_sc_mesh(),
        out_type=jax.ShapeDtypeStruct((A_PAD, HH), f32),
        compiler_params=_SC_PARAMS,
        scratch_types=[
            pltpu.VMEM((_ROWS, HH), f32),
            pltpu.VMEM((E_PAD + 16,), i32),
            pltpu.VMEM((16,), i32),
            pltpu.VMEM((G,), i32),
            pltpu.VMEM((G,), i32),
            pltpu.VMEM((G, HH), f32),
            pltpu.VMEM((G, HH), f32),
            pltpu.SemaphoreType.DMA,
            pltpu.SemaphoreType.DMA,
        ],
    )
    def k(m_hbm, hits_hbm, cnt_hbm, agg_hbm, acc_v, hits_v, cnt_v,
          idxa_v, idxb_v, rowsa_v, rowsb_v, sema, semb):
        wid = lax.axis_index("s") * 2 + lax.axis_index("c")
        lo = wid * _ROWS
        lane = lax.broadcasted_iota(i32, (16,), 0)

        pltpu.sync_copy(cnt_hbm.at[wid], cnt_v)
        cnt = cnt_v[...][0]
        for c in range(E_PAD // _HCH):
            @pl.when(c * _HCH < cnt)
            def _():
                pltpu.sync_copy(hits_hbm.at[wid, pl.ds(c * _HCH, _HCH)],
                                hits_v.at[pl.ds(c * _HCH, _HCH)])

        neginf = jnp.full((16,), -jnp.inf, f32)

        def init_row(r, _):
            for c in range(HH // 16):
                acc_v[r, pl.ds(c * 16, 16)] = neginf
            return 0

        lax.fori_loop(0, _ROWS, init_row, 0, unroll=False)

        ng = (cnt + G - 1) // G

        def stage_and_start(g, idxt_v, rows_v, sem):
            base = g * G
            for vv in range(G // 16):
                pos = base + vv * 16
                p = hits_v[pl.ds(pos, 16)]
                valid = (pos + lane) < cnt
                idxt_v[pl.ds(vv * 16, 16)] = jnp.where(valid, p & 0xFFFF, 0)
            return pltpu.async_copy(m_hbm.at[idxt_v], rows_v, sem)

        def accum(g, rows_v):
            base = g * G
            count = jnp.minimum(G, cnt - base)

            def rbody(r, _):
                pk = hits_v[pl.ds(base + r, 16)][0]
                off = pk >> 16

                @pl.when(r < count)
                def _():
                    for c in range(HH // 16):
                        sl = pl.ds(c * 16, 16)
                        acc_v[off, sl] = jnp.maximum(acc_v[off, sl], rows_v[r, sl])

                return 0

            lax.fori_loop(0, G, rbody, 0, unroll=False)

        @pl.when(ng > 0)
        def _():
            stage_and_start(0, idxa_v, rowsa_v, sema)

        npair = (ng + 1) // 2

        def pbody(p, _):
            e = 2 * p
            o = e + 1

            @pl.when(o < ng)
            def _():
                stage_and_start(o, idxb_v, rowsb_v, semb)

            pltpu.make_async_copy(m_hbm.at[idxa_v], rowsa_v, sema).wait()
            accum(e, rowsa_v)

            @pl.when(o + 1 < ng)
            def _():
                stage_and_start(o + 1, idxa_v, rowsa_v, sema)

            @pl.when(o < ng)
            def _():
                pltpu.make_async_copy(m_hbm.at[idxb_v], rowsb_v, semb).wait()
                accum(o, rowsb_v)

            return 0

        lax.fori_loop(0, npair, pbody, 0, unroll=False)

        pltpu.sync_copy(acc_v, agg_hbm.at[pl.ds(lo, _ROWS)])

    return k(m, hits, cnt)


def _mm(x, w):
    def body(x_ref, w_ref, o_ref):
        o_ref[...] = jnp.dot(x_ref[...], w_ref[...], preferred_element_type=f32)

    return pl.pallas_call(
        body,
        out_shape=jax.ShapeDtypeStruct((x.shape[0], w.shape[1]), f32),
    )(x, w)


def _edge_init(attr, w1, b1, w2, b2):
    R = 2048
    grid = E_PAD // R

    def body(a_ref, w1_ref, b1_ref, w2_ref, b2_ref, o_ref):
        h = jnp.maximum(
            jnp.dot(a_ref[...], w1_ref[...], preferred_element_type=f32)
            + b1_ref[...], 0.0)
        o_ref[...] = jnp.dot(h, w2_ref[...], preferred_element_type=f32) + b2_ref[...]

    full = lambda s: pl.BlockSpec(s, lambda i: (0, 0))
    return pl.pallas_call(
        body,
        grid=(grid,),
        in_specs=[
            pl.BlockSpec((R, 16), lambda i: (i, 0)),
            full((16, HH)), full((1, HH)), full((HH, HH)), full((1, HH)),
        ],
        out_specs=pl.BlockSpec((R, HH), lambda i: (i, 0)),
        out_shape=jax.ShapeDtypeStruct((E_PAD, HH), f32),
    )(attr, w1, b1.reshape(1, HH), w2, b2.reshape(1, HH))


def _edge_mlp(ea, g0, u0, g1, u1, w1e, b1, w2, b2, wf1, bf1, wf2, bf2):
    R = 2048
    grid = E_PAD // R
    k0 = g0.shape[1]
    k1 = g1.shape[1]

    def body(ea_ref, g0_ref, g1_ref, *refs):
        i = 0
        if u0 is not None:
            u0_ref = refs[i]; i += 1
        if u1 is not None:
            u1_ref = refs[i]; i += 1
        (w1e_ref, b1_ref, w2_ref, b2_ref,
         wf1_ref, bf1_ref, wf2_ref, bf2_ref, ean_ref, m_ref) = refs[i:]
        a = ea_ref[...]
        p0 = (jnp.dot(g0_ref[...], u0_ref[...], preferred_element_type=f32)
              if u0 is not None else g0_ref[...])
        p1 = (jnp.dot(g1_ref[...], u1_ref[...], preferred_element_type=f32)
              if u1 is not None else g1_ref[...])
        pre = (jnp.dot(a, w1e_ref[...], preferred_element_type=f32)
               + p0 + p1 + b1_ref[...])
        h = jnp.maximum(pre, 0.0)
        ean = a + jnp.dot(h, w2_ref[...], preferred_element_type=f32) + b2_ref[...]
        ean_ref[...] = ean
        h2 = jnp.maximum(
            jnp.dot(ean, wf1_ref[...], preferred_element_type=f32) + bf1_ref[...], 0.0)
        m_ref[...] = jnp.dot(h2, wf2_ref[...], preferred_element_type=f32) + bf2_ref[...]

    eb = pl.BlockSpec((R, HH), lambda i: (i, 0))
    full = lambda s: pl.BlockSpec(s, lambda i: (0, 0))
    ins = [ea, g0, g1]
    specs = [eb, pl.BlockSpec((R, k0), lambda i: (i, 0)),
             pl.BlockSpec((R, k1), lambda i: (i, 0))]
    if u0 is not None:
        ins.append(u0); specs.append(full((k0, HH)))
    if u1 is not None:
        ins.append(u1); specs.append(full((k1, HH)))
    ins += [w1e, b1.reshape(1, HH), w2, b2.reshape(1, HH),
            wf1, bf1.reshape(1, HH), wf2, bf2.reshape(1, HH)]
    specs += [full((HH, HH)), full((1, HH)), full((HH, HH)), full((1, HH)),
              full((HH, HH)), full((1, HH)), full((HH, HH)), full((1, HH))]
    return pl.pallas_call(
        body,
        grid=(grid,),
        in_specs=specs,
        out_specs=[eb, eb],
        out_shape=[jax.ShapeDtypeStruct((E_PAD, HH), f32),
                   jax.ShapeDtypeStruct((E_PAD, HH), f32)],
    )(*ins)


def _agent_update(xa, a0, a1, a2):
    def body(x_ref, a0_ref, a1_ref, a2_ref, o_ref):
        def fix(v):
            return jnp.where(jnp.isfinite(v), v, 0.0)
        o_ref[...] = x_ref[...] + jnp.maximum(
            jnp.maximum(fix(a0_ref[...]), fix(a1_ref[...])), fix(a2_ref[...]))

    return pl.pallas_call(
        body,
        out_shape=jax.ShapeDtypeStruct((A_PAD, HH), f32),
    )(xa, a0, a1, a2)


def _field(xa, act, w1v, w1a, b1, w2, b2):
    def body(x_ref, a_ref, w1v_ref, w1a_ref, b1_ref, w2_ref, b2_ref, o_ref):
        h = jnp.maximum(
            jnp.dot(x_ref[...], w1v_ref[...], preferred_element_type=f32)
            + jnp.dot(a_ref[...], w1a_ref[...], preferred_element_type=f32)
            + b1_ref[...], 0.0)
        o_ref[...] = jnp.dot(h, w2_ref[...], preferred_element_type=f32) + b2_ref[...]

    return pl.pallas_call(
        body,
        out_shape=jax.ShapeDtypeStruct((A_PAD, 1), f32),
    )(xa, act, w1v, w1a, b1.reshape(1, HH), w2, b2.reshape(1, 1))


def kernel(x_obstacle, x_agent, x_goal, action,
           edge_index_oa, edge_attr_oa,
           edge_index_aa, edge_attr_aa,
           edge_index_ga, edge_attr_ga, params):
    types = ["oa", "aa", "ga"]
    ei = {"oa": edge_index_oa, "aa": edge_index_aa, "ga": edge_index_ga}
    eattr = {"oa": edge_attr_oa, "aa": edge_attr_aa, "ga": edge_attr_ga}

    xo_raw = _padr(x_obstacle, O_PAD)
    xa_raw = _padr(x_agent, A_PAD)
    xg_raw = _padr(x_goal, G_PAD)
    W_emb = params["W_embed"]
    xo = _mm(xo_raw, W_emb)
    xa = _mm(xa_raw, W_emb)
    xg = _mm(xg_raw, W_emb)
    act = _padr(action, A_PAD)

    ea, idx0, idx1 = {}, {}, {}
    for t in types:
        pp = params["ee_" + t]
        ea[t] = _edge_init(_padr(eattr[t], E_PAD), pp["W1"], pp["b1"], pp["W2"], pp["b2"])
        e = ei[t]
        idx0[t] = jnp.pad(e[0], (0, E_PAD - e.shape[1]), constant_values=0).astype(i32)
        idx1[t] = jnp.pad(e[1], (0, E_PAD - e.shape[1]), constant_values=0).astype(i32)

    hits, cnts = {}, {}
    for t in types:
        dst = jnp.pad(ei[t][1], (0, E_PAD - ei[t].shape[1]),
                      constant_values=SENTINEL).astype(i32)
        hits[t], cnts[t] = _sc_hits(dst)

    raw_src_tab = {"oa": xo_raw, "aa": xa_raw, "ga": xg_raw}
    rx0 = {t: _sc_gather(raw_src_tab[t], idx0[t]) for t in types}
    rx1 = {t: _sc_gather(xa_raw, idx1[t]) for t in types}

    for l in range(2):
        aggs = []
        for t in types:
            em = params["em_%d_%s" % (l, t)]
            W1 = em["W1"]
            W1e, W1s, W1d = W1[:HH], W1[HH:2 * HH], W1[2 * HH:]
            if l == 0:
                g0, u0 = rx0[t], _mm(W_emb, W1s)
                g1, u1 = rx1[t], _mm(W_emb, W1d)
            else:
                if t == "aa":
                    g0, u0 = _sc_gather(_mm(xa, W1s), idx0[t]), None
                else:
                    g0, u0 = rx0[t], _mm(W_emb, W1s)
                g1, u1 = _sc_gather(_mm(xa, W1d), idx1[t]), None
            fx = params["fx_%d_%s" % (l, t)]
            ea[t], m = _edge_mlp(ea[t], g0, u0, g1, u1,
                                 W1e, em["b1"], em["W2"], em["b2"],
                                 fx["W1"], fx["b1"], fx["W2"], fx["b2"])
            aggs.append(_sc_segmax(m, hits[t], cnts[t]))
        xa = _agent_update(xa, aggs[0], aggs[1], aggs[2])

    fp = params["field"]
    W1f = fp["W1"]
    out = _field(xa, act, W1f[:HH], W1f[HH:], fp["b1"], fp["W2"], fp["b2"])
    return out[:x_agent.shape[0], 0]


# speedup vs baseline: 1.3508x; 1.0270x over previous
import functools

import jax
import jax.numpy as jnp
from jax import lax
from jax.experimental import pallas as pl
from jax.experimental.pallas import tpu as pltpu
from jax.experimental.pallas import tpu_sc as plsc

HH = 256
E_PAD = 53248
A_PAD = 5120
O_PAD = 4096
G_PAD = 1024
NW = 32
SENTINEL = 1 << 20

f32 = jnp.float32
i32 = jnp.int32


def _padr(x, n, val=0.0):
    pads = ((0, n - x.shape[0]),) + ((0, 0),) * (x.ndim - 1)
    return jnp.pad(x, pads, constant_values=val)


def _sc_mesh():
    return plsc.VectorSubcoreMesh(core_axis_name="c", subcore_axis_name="s")


_SC_PARAMS = pltpu.CompilerParams(needs_layout_passes=False)


def _sc_gather_multi(tables, idxs):
    W = tables[0].shape[1]
    assert all(t.shape[1] == W for t in tables)
    nj = len(tables)
    per_w = E_PAD // NW
    C = 128
    nch = per_w // C
    NB = 3

    @functools.partial(
        pl.kernel,
        mesh=_sc_mesh(),
        out_type=[jax.ShapeDtypeStruct((E_PAD, W), f32)] * nj,
        compiler_params=_SC_PARAMS,
        scratch_types=[pltpu.VMEM((per_w,), i32)]
        + [pltpu.VMEM((C, W), f32)] * NB
        + [pltpu.SemaphoreType.DMA] * NB
        + [pltpu.SemaphoreType.DMA] * NB,
    )
    def k(*refs):
        tabs = refs[:nj]
        idxr = refs[nj:2 * nj]
        outs = refs[2 * nj:3 * nj]
        idx_v = refs[3 * nj]
        rows = refs[3 * nj + 1:3 * nj + 1 + NB]
        gsem = refs[3 * nj + 1 + NB:3 * nj + 1 + 2 * NB]
        wsem = refs[3 * nj + 1 + 2 * NB:]
        wid = lax.axis_index("s") * 2 + lax.axis_index("c")
        base0 = wid * per_w

        for j in range(nj):
            pltpu.sync_copy(idxr[j].at[pl.ds(base0, per_w)], idx_v)
            gathers = [None] * nch
            writes = [None] * nch
            for kk in range(nch):
                b = kk % NB
                if kk >= NB:
                    writes[kk - NB].wait()
                gathers[kk] = pltpu.async_copy(
                    tabs[j].at[idx_v.at[pl.ds(kk * C, C)]], rows[b], gsem[b])
                if kk >= 1:
                    gathers[kk - 1].wait()
                    writes[kk - 1] = pltpu.async_copy(
                        rows[(kk - 1) % NB],
                        outs[j].at[pl.ds(base0 + (kk - 1) * C, C)],
                        wsem[(kk - 1) % NB])
            gathers[nch - 1].wait()
            writes[nch - 1] = pltpu.async_copy(
                rows[(nch - 1) % NB],
                outs[j].at[pl.ds(base0 + (nch - 1) * C, C)],
                wsem[(nch - 1) % NB])
            for kk in range(nch - NB, nch):
                writes[kk].wait()

    return k(*tables, *idxs)


_ROWS = A_PAD // NW
_HCH = 4096


def _sc_hits3(dsts):
    DCH = 2048
    nch = E_PAD // DCH
    nj = len(dsts)

    @functools.partial(
        pl.kernel,
        mesh=_sc_mesh(),
        out_type=[jax.ShapeDtypeStruct((NW, E_PAD), i32)] * nj
        + [jax.ShapeDtypeStruct((NW, 16), i32)] * nj,
        compiler_params=_SC_PARAMS,
        scratch_types=[
            pltpu.VMEM((DCH,), i32),
            pltpu.VMEM((E_PAD + 16,), i32),
            pltpu.VMEM((16,), i32),
        ],
    )
    def k(*refs):
        dst_hbms = refs[:nj]
        hits_hbms = refs[nj:2 * nj]
        cnt_hbms = refs[2 * nj:3 * nj]
        dbuf_v, hits_v, cnt_v = refs[3 * nj:]
        wid = lax.axis_index("s") * 2 + lax.axis_index("c")
        lo = wid * _ROWS
        hi = lo + _ROWS
        lane = lax.broadcasted_iota(i32, (16,), 0)

        for j in range(nj):
            def chunk_body(kk, cnt, dst_hbm=dst_hbms[j]):
                pltpu.sync_copy(dst_hbm.at[pl.ds(kk * DCH, DCH)], dbuf_v)

                def vbody(v, cnt):
                    d = dbuf_v[pl.ds(v * 16, 16)]
                    msk = (d >= lo) & (d < hi)
                    eid = kk * DCH + v * 16 + lane
                    packed = ((d - lo) << 16) | eid
                    pos = plsc.cumsum(msk.astype(i32))
                    plsc.store_scatter(hits_v, [cnt + pos - 1], packed, mask=msk)
                    return cnt + pos[15]

                return lax.fori_loop(0, DCH // 16, vbody, cnt, unroll=False)

            cnt = lax.fori_loop(0, nch, chunk_body, 0, unroll=False)

            cnt_v[...] = jnp.zeros((16,), i32) + cnt
            pltpu.sync_copy(cnt_v, cnt_hbms[j].at[wid])
            for c in range(E_PAD // _HCH):
                @pl.when(c * _HCH < cnt)
                def _(c=c, j=j):
                    pltpu.sync_copy(hits_v.at[pl.ds(c * _HCH, _HCH)],
                                    hits_hbms[j].at[wid, pl.ds(c * _HCH, _HCH)])

    outs = k(*dsts)
    return outs[:nj], outs[nj:]


def _sc_segmax3(ms, hits, cnts):
    G = 64
    nj = len(ms)

    @functools.partial(
        pl.kernel,
        mesh=_sc_mesh(),
        out_type=[jax.ShapeDtypeStruct((A_PAD, HH), f32)] * nj,
        compiler_params=_SC_PARAMS,
        scratch_types=[
            pltpu.VMEM((_ROWS, HH), f32),
            pltpu.VMEM((E_PAD + 16,), i32),
            pltpu.VMEM((16,), i32),
            pltpu.VMEM((G,), i32),
            pltpu.VMEM((G,), i32),
            pltpu.VMEM((G, HH), f32),
            pltpu.VMEM((G, HH), f32),
            pltpu.SemaphoreType.DMA,
            pltpu.SemaphoreType.DMA,
        ],
    )
    def k(*refs):
        m_hbms = refs[:nj]
        hits_hbms = refs[nj:2 * nj]
        cnt_hbms = refs[2 * nj:3 * nj]
        agg_hbms = refs[3 * nj:4 * nj]
        (acc_v, hits_v, cnt_v, idxa_v, idxb_v,
         rowsa_v, rowsb_v, sema, semb) = refs[4 * nj:]
        wid = lax.axis_index("s") * 2 + lax.axis_index("c")
        lo = wid * _ROWS
        lane = lax.broadcasted_iota(i32, (16,), 0)
        neginf = jnp.full((16,), -jnp.inf, f32)

        for j in range(nj):
            m_hbm = m_hbms[j]
            pltpu.sync_copy(cnt_hbms[j].at[wid], cnt_v)
            cnt = cnt_v[...][0]
            for c in range(E_PAD // _HCH):
                @pl.when(c * _HCH < cnt)
                def _(c=c, j=j):
                    pltpu.sync_copy(hits_hbms[j].at[wid, pl.ds(c * _HCH, _HCH)],
                                    hits_v.at[pl.ds(c * _HCH, _HCH)])

            def init_row(r, _):
                for c in range(HH // 16):
                    acc_v[r, pl.ds(c * 16, 16)] = neginf
                return 0

            lax.fori_loop(0, _ROWS, init_row, 0, unroll=False)

            ng = (cnt + G - 1) // G

            def stage_and_start(g, idxt_v, rows_v, sem):
                base = g * G
                for vv in range(G // 16):
                    pos = base + vv * 16
                    p = hits_v[pl.ds(pos, 16)]
                    valid = (pos + lane) < cnt
                    idxt_v[pl.ds(vv * 16, 16)] = jnp.where(valid, p & 0xFFFF, 0)
                return pltpu.async_copy(m_hbm.at[idxt_v], rows_v, sem)

            def accum(g, rows_v):
                base = g * G
                count = jnp.minimum(G, cnt - base)

                def rbody(r, _):
                    pk = hits_v[pl.ds(base + r, 16)][0]
                    off = pk >> 16

                    @pl.when(r < count)
                    def _():
                        for c in range(HH // 16):
                            sl = pl.ds(c * 16, 16)
                            acc_v[off, sl] = jnp.maximum(acc_v[off, sl],
                                                         rows_v[r, sl])

                    return 0

                lax.fori_loop(0, G, rbody, 0, unroll=False)

            @pl.when(ng > 0)
            def _():
                stage_and_start(0, idxa_v, rowsa_v, sema)

            npair = (ng + 1) // 2

            def pbody(p, _):
                e = 2 * p
                o = e + 1

                @pl.when(o < ng)
                def _():
                    stage_and_start(o, idxb_v, rowsb_v, semb)

                pltpu.make_async_copy(m_hbm.at[idxa_v], rowsa_v, sema).wait()
                accum(e, rowsa_v)

                @pl.when(o + 1 < ng)
                def _():
                    stage_and_start(o + 1, idxa_v, rowsa_v, sema)

                @pl.when(o < ng)
                def _():
                    pltpu.make_async_copy(m_hbm.at[idxb_v], rowsb_v, semb).wait()
                    accum(o, rowsb_v)

                return 0

            lax.fori_loop(0, npair, pbody, 0, unroll=False)

            pltpu.sync_copy(acc_v, agg_hbms[j].at[pl.ds(lo, _ROWS)])

    return k(*ms, *hits, *cnts)


def _mm(x, w):
    def body(x_ref, w_ref, o_ref):
        o_ref[...] = jnp.dot(x_ref[...], w_ref[...], preferred_element_type=f32)

    return pl.pallas_call(
        body,
        out_shape=jax.ShapeDtypeStruct((x.shape[0], w.shape[1]), f32),
    )(x, w)


def _edge_init(attr, w1, b1, w2, b2):
    R = 2048
    grid = E_PAD // R

    def body(a_ref, w1_ref, b1_ref, w2_ref, b2_ref, o_ref):
        h = jnp.maximum(
            jnp.dot(a_ref[...], w1_ref[...], preferred_element_type=f32)
            + b1_ref[...], 0.0)
        o_ref[...] = jnp.dot(h, w2_ref[...], preferred_element_type=f32) + b2_ref[...]

    full = lambda s: pl.BlockSpec(s, lambda i: (0, 0))
    return pl.pallas_call(
        body,
        grid=(grid,),
        in_specs=[
            pl.BlockSpec((R, 16), lambda i: (i, 0)),
            full((16, HH)), full((1, HH)), full((HH, HH)), full((1, HH)),
        ],
        out_specs=pl.BlockSpec((R, HH), lambda i: (i, 0)),
        out_shape=jax.ShapeDtypeStruct((E_PAD, HH), f32),
    )(attr, w1, b1.reshape(1, HH), w2, b2.reshape(1, HH))


def _edge_mlp(ea, g0, u0, g1, u1, w1e, b1, w2, b2, wf1, bf1, wf2, bf2):
    R = 2048
    grid = E_PAD // R
    k0 = g0.shape[1]
    k1 = g1.shape[1]

    def body(ea_ref, g0_ref, g1_ref, *refs):
        i = 0
        if u0 is not None:
            u0_ref = refs[i]; i += 1
        if u1 is not None:
            u1_ref = refs[i]; i += 1
        (w1e_ref, b1_ref, w2_ref, b2_ref,
         wf1_ref, bf1_ref, wf2_ref, bf2_ref, ean_ref, m_ref) = refs[i:]
        a = ea_ref[...]
        p0 = (jnp.dot(g0_ref[...], u0_ref[...], preferred_element_type=f32)
              if u0 is not None else g0_ref[...])
        p1 = (jnp.dot(g1_ref[...], u1_ref[...], preferred_element_type=f32)
              if u1 is not None else g1_ref[...])
        pre = (jnp.dot(a, w1e_ref[...], preferred_element_type=f32)
               + p0 + p1 + b1_ref[...])
        h = jnp.maximum(pre, 0.0)
        ean = a + jnp.dot(h, w2_ref[...], preferred_element_type=f32) + b2_ref[...]
        ean_ref[...] = ean
        h2 = jnp.maximum(
            jnp.dot(ean, wf1_ref[...], preferred_element_type=f32) + bf1_ref[...], 0.0)
        m_ref[...] = jnp.dot(h2, wf2_ref[...], preferred_element_type=f32) + bf2_ref[...]

    eb = pl.BlockSpec((R, HH), lambda i: (i, 0))
    full = lambda s: pl.BlockSpec(s, lambda i: (0, 0))
    ins = [ea, g0, g1]
    specs = [eb, pl.BlockSpec((R, k0), lambda i: (i, 0)),
             pl.BlockSpec((R, k1), lambda i: (i, 0))]
    if u0 is not None:
        ins.append(u0); specs.append(full((k0, HH)))
    if u1 is not None:
        ins.append(u1); specs.append(full((k1, HH)))
    ins += [w1e, b1.reshape(1, HH), w2, b2.reshape(1, HH),
            wf1, bf1.reshape(1, HH), wf2, bf2.reshape(1, HH)]
    specs += [full((HH, HH)), full((1, HH)), full((HH, HH)), full((1, HH)),
              full((HH, HH)), full((1, HH)), full((HH, HH)), full((1, HH))]
    return pl.pallas_call(
        body,
        grid=(grid,),
        in_specs=specs,
        out_specs=[eb, eb],
        out_shape=[jax.ShapeDtypeStruct((E_PAD, HH), f32),
                   jax.ShapeDtypeStruct((E_PAD, HH), f32)],
    )(*ins)


def _agent_update(xa, a0, a1, a2):
    def body(x_ref, a0_ref, a1_ref, a2_ref, o_ref):
        def fix(v):
            return jnp.where(jnp.isfinite(v), v, 0.0)
        o_ref[...] = x_ref[...] + jnp.maximum(
            jnp.maximum(fix(a0_ref[...]), fix(a1_ref[...])), fix(a2_ref[...]))

    return pl.pallas_call(
        body,
        out_shape=jax.ShapeDtypeStruct((A_PAD, HH), f32),
    )(xa, a0, a1, a2)


def _field(xa, act, w1v, w1a, b1, w2, b2):
    def body(x_ref, a_ref, w1v_ref, w1a_ref, b1_ref, w2_ref, b2_ref, o_ref):
        h = jnp.maximum(
            jnp.dot(x_ref[...], w1v_ref[...], preferred_element_type=f32)
            + jnp.dot(a_ref[...], w1a_ref[...], preferred_element_type=f32)
            + b1_ref[...], 0.0)
        o_ref[...] = jnp.dot(h, w2_ref[...], preferred_element_type=f32) + b2_ref[...]

    return pl.pallas_call(
        body,
        out_shape=jax.ShapeDtypeStruct((A_PAD, 1), f32),
    )(xa, act, w1v, w1a, b1.reshape(1, HH), w2, b2.reshape(1, 1))


def kernel(x_obstacle, x_agent, x_goal, action,
           edge_index_oa, edge_attr_oa,
           edge_index_aa, edge_attr_aa,
           edge_index_ga, edge_attr_ga, params):
    types = ["oa", "aa", "ga"]
    ei = {"oa": edge_index_oa, "aa": edge_index_aa, "ga": edge_index_ga}
    eattr = {"oa": edge_attr_oa, "aa": edge_attr_aa, "ga": edge_attr_ga}

    xo_raw = _padr(x_obstacle, O_PAD)
    xa_raw = _padr(x_agent, A_PAD)
    xg_raw = _padr(x_goal, G_PAD)
    W_emb = params["W_embed"]
    xo = _mm(xo_raw, W_emb)
    xa = _mm(xa_raw, W_emb)
    xg = _mm(xg_raw, W_emb)
    act = _padr(action, A_PAD)

    ea, idx0, idx1 = {}, {}, {}
    for t in types:
        pp = params["ee_" + t]
        ea[t] = _edge_init(_padr(eattr[t], E_PAD), pp["W1"], pp["b1"], pp["W2"], pp["b2"])
        e = ei[t]
        idx0[t] = jnp.pad(e[0], (0, E_PAD - e.shape[1]), constant_values=0).astype(i32)
        idx1[t] = jnp.pad(e[1], (0, E_PAD - e.shape[1]), constant_values=0).astype(i32)

    dsts = [jnp.pad(ei[t][1], (0, E_PAD - ei[t].shape[1]),
                    constant_values=SENTINEL).astype(i32) for t in types]
    hits_l, cnts_l = _sc_hits3(dsts)

    r6 = _sc_gather_multi(
        [xo_raw, xa_raw, xg_raw, xa_raw, xa_raw, xa_raw],
        [idx0["oa"], idx0["aa"], idx0["ga"],
         idx1["oa"], idx1["aa"], idx1["ga"]])
    rx0 = dict(zip(types, r6[:3]))
    rx1 = dict(zip(types, r6[3:]))

    def w1split(l, t):
        W1 = params["em_%d_%s" % (l, t)]["W1"]
        return W1[:HH], W1[HH:2 * HH], W1[2 * HH:]

    for l in range(2):
        gathered = {}
        if l == 1:
            _, W1s_aa, _ = w1split(1, "aa")
            projs = [_mm(xa, W1s_aa)] + [_mm(xa, w1split(1, t)[2])
                                         for t in types]
            g4 = _sc_gather_multi(
                projs, [idx0["aa"], idx1["oa"], idx1["aa"], idx1["ga"]])
            gathered = {"aa_src": g4[0], "oa_dst": g4[1],
                        "aa_dst": g4[2], "ga_dst": g4[3]}
        ms = []
        for t in types:
            em = params["em_%d_%s" % (l, t)]
            W1e, W1s, W1d = w1split(l, t)
            if l == 0:
                g0, u0 = rx0[t], _mm(W_emb, W1s)
                g1, u1 = rx1[t], _mm(W_emb, W1d)
            else:
                if t == "aa":
                    g0, u0 = gathered["aa_src"], None
                else:
                    g0, u0 = rx0[t], _mm(W_emb, W1s)
                g1, u1 = gathered[t + "_dst"], None
            fx = params["fx_%d_%s" % (l, t)]
            ea[t], m = _edge_mlp(ea[t], g0, u0, g1, u1,
                                 W1e, em["b1"], em["W2"], em["b2"],
                                 fx["W1"], fx["b1"], fx["W2"], fx["b2"])
            ms.append(m)
        aggs = _sc_segmax3(ms, hits_l, cnts_l)
        xa = _agent_update(xa, aggs[0], aggs[1], aggs[2])

    fp = params["field"]
    W1f = fp["W1"]
    out = _field(xa, act, W1f[:HH], W1f[HH:], fp["b1"], fp["W2"], fp["b2"])
    return out[:x_agent.shape[0], 0]


# speedup vs baseline: 1.3514x; 1.0005x over previous
import functools

import jax
import jax.numpy as jnp
from jax import lax
from jax.experimental import pallas as pl
from jax.experimental.pallas import tpu as pltpu
from jax.experimental.pallas import tpu_sc as plsc

HH = 256
E_PAD = 53248
A_PAD = 5120
O_PAD = 4096
G_PAD = 1024
NW = 32
SENTINEL = 1 << 20

f32 = jnp.float32
i32 = jnp.int32


def _padr(x, n, val=0.0):
    pads = ((0, n - x.shape[0]),) + ((0, 0),) * (x.ndim - 1)
    return jnp.pad(x, pads, constant_values=val)


def _sc_mesh():
    return plsc.VectorSubcoreMesh(core_axis_name="c", subcore_axis_name="s")


_SC_PARAMS = pltpu.CompilerParams(needs_layout_passes=False)


def _sc_gather_multi(tables, idxs):
    W = tables[0].shape[1]
    assert all(t.shape[1] == W for t in tables)
    nj = len(tables)
    per_w = E_PAD // NW
    C = 128 if W <= 128 else 64
    nch = per_w // C
    NB = 4
    D = 2

    @functools.partial(
        pl.kernel,
        mesh=_sc_mesh(),
        out_type=[jax.ShapeDtypeStruct((E_PAD, W), f32)] * nj,
        compiler_params=_SC_PARAMS,
        scratch_types=[pltpu.VMEM((per_w,), i32)]
        + [pltpu.VMEM((C, W), f32)] * NB
        + [pltpu.SemaphoreType.DMA] * NB
        + [pltpu.SemaphoreType.DMA] * NB,
    )
    def k(*refs):
        tabs = refs[:nj]
        idxr = refs[nj:2 * nj]
        outs = refs[2 * nj:3 * nj]
        idx_v = refs[3 * nj]
        rows = refs[3 * nj + 1:3 * nj + 1 + NB]
        gsem = refs[3 * nj + 1 + NB:3 * nj + 1 + 2 * NB]
        wsem = refs[3 * nj + 1 + 2 * NB:]
        wid = lax.axis_index("s") * 2 + lax.axis_index("c")
        base0 = wid * per_w

        for j in range(nj):
            pltpu.sync_copy(idxr[j].at[pl.ds(base0, per_w)], idx_v)
            gathers = [None] * nch
            writes = [None] * nch

            def start_gather(kk):
                b = kk % NB
                if kk >= NB:
                    writes[kk - NB].wait()
                gathers[kk] = pltpu.async_copy(
                    tabs[j].at[idx_v.at[pl.ds(kk * C, C)]], rows[b], gsem[b])

            for kk in range(min(D, nch)):
                start_gather(kk)
            for kk in range(nch):
                if kk + D < nch:
                    start_gather(kk + D)
                gathers[kk].wait()
                writes[kk] = pltpu.async_copy(
                    rows[kk % NB],
                    outs[j].at[pl.ds(base0 + kk * C, C)],
                    wsem[kk % NB])
            for kk in range(max(0, nch - NB), nch):
                writes[kk].wait()

    return k(*tables, *idxs)


_ROWS = A_PAD // NW
_HCH = 4096


def _sc_hits3(dsts):
    DCH = 2048
    nch = E_PAD // DCH
    nj = len(dsts)

    @functools.partial(
        pl.kernel,
        mesh=_sc_mesh(),
        out_type=[jax.ShapeDtypeStruct((NW, E_PAD), i32)] * nj
        + [jax.ShapeDtypeStruct((NW, 16), i32)] * nj,
        compiler_params=_SC_PARAMS,
        scratch_types=[
            pltpu.VMEM((DCH,), i32),
            pltpu.VMEM((E_PAD + 16,), i32),
            pltpu.VMEM((16,), i32),
        ],
    )
    def k(*refs):
        dst_hbms = refs[:nj]
        hits_hbms = refs[nj:2 * nj]
        cnt_hbms = refs[2 * nj:3 * nj]
        dbuf_v, hits_v, cnt_v = refs[3 * nj:]
        wid = lax.axis_index("s") * 2 + lax.axis_index("c")
        lo = wid * _ROWS
        hi = lo + _ROWS
        lane = lax.broadcasted_iota(i32, (16,), 0)

        for j in range(nj):
            def chunk_body(kk, cnt, dst_hbm=dst_hbms[j]):
                pltpu.sync_copy(dst_hbm.at[pl.ds(kk * DCH, DCH)], dbuf_v)

                def vbody(v, cnt):
                    d = dbuf_v[pl.ds(v * 16, 16)]
                    msk = (d >= lo) & (d < hi)
                    eid = kk * DCH + v * 16 + lane
                    packed = ((d - lo) << 16) | eid
                    pos = plsc.cumsum(msk.astype(i32))
                    plsc.store_scatter(hits_v, [cnt + pos - 1], packed, mask=msk)
                    return cnt + pos[15]

                return lax.fori_loop(0, DCH // 16, vbody, cnt, unroll=False)

            cnt = lax.fori_loop(0, nch, chunk_body, 0, unroll=False)

            cnt_v[...] = jnp.zeros((16,), i32) + cnt
            pltpu.sync_copy(cnt_v, cnt_hbms[j].at[wid])
            for c in range(E_PAD // _HCH):
                @pl.when(c * _HCH < cnt)
                def _(c=c, j=j):
                    pltpu.sync_copy(hits_v.at[pl.ds(c * _HCH, _HCH)],
                                    hits_hbms[j].at[wid, pl.ds(c * _HCH, _HCH)])

    outs = k(*dsts)
    return outs[:nj], outs[nj:]


def _sc_segmax3(ms, hits, cnts):
    G = 64
    nj = len(ms)

    @functools.partial(
        pl.kernel,
        mesh=_sc_mesh(),
        out_type=[jax.ShapeDtypeStruct((A_PAD, HH), f32)] * nj,
        compiler_params=_SC_PARAMS,
        scratch_types=[
            pltpu.VMEM((_ROWS, HH), f32),
            pltpu.VMEM((E_PAD + 16,), i32),
            pltpu.VMEM((16,), i32),
            pltpu.VMEM((G,), i32),
            pltpu.VMEM((G,), i32),
            pltpu.VMEM((G, HH), f32),
            pltpu.VMEM((G, HH), f32),
            pltpu.SemaphoreType.DMA,
            pltpu.SemaphoreType.DMA,
        ],
    )
    def k(*refs):
        m_hbms = refs[:nj]
        hits_hbms = refs[nj:2 * nj]
        cnt_hbms = refs[2 * nj:3 * nj]
        agg_hbms = refs[3 * nj:4 * nj]
        (acc_v, hits_v, cnt_v, idxa_v, idxb_v,
         rowsa_v, rowsb_v, sema, semb) = refs[4 * nj:]
        wid = lax.axis_index("s") * 2 + lax.axis_index("c")
        lo = wid * _ROWS
        lane = lax.broadcasted_iota(i32, (16,), 0)
        neginf = jnp.full((16,), -jnp.inf, f32)

        for j in range(nj):
            m_hbm = m_hbms[j]
            pltpu.sync_copy(cnt_hbms[j].at[wid], cnt_v)
            cnt = cnt_v[...][0]
            for c in range(E_PAD // _HCH):
                @pl.when(c * _HCH < cnt)
                def _(c=c, j=j):
                    pltpu.sync_copy(hits_hbms[j].at[wid, pl.ds(c * _HCH, _HCH)],
                                    hits_v.at[pl.ds(c * _HCH, _HCH)])

            def init_row(r, _):
                for c in range(HH // 16):
                    acc_v[r, pl.ds(c * 16, 16)] = neginf
                return 0

            lax.fori_loop(0, _ROWS, init_row, 0, unroll=False)

            ng = (cnt + G - 1) // G

            def stage_and_start(g, idxt_v, rows_v, sem):
                base = g * G
                for vv in range(G // 16):
                    pos = base + vv * 16
                    p = hits_v[pl.ds(pos, 16)]
                    valid = (pos + lane) < cnt
                    idxt_v[pl.ds(vv * 16, 16)] = jnp.where(valid, p & 0xFFFF, 0)
                return pltpu.async_copy(m_hbm.at[idxt_v], rows_v, sem)

            def accum(g, rows_v):
                base = g * G
                count = jnp.minimum(G, cnt - base)

                def rbody(r, _):
                    pk = hits_v[pl.ds(base + r, 16)][0]
                    off = pk >> 16

                    @pl.when(r < count)
                    def _():
                        for c in range(HH // 16):
                            sl = pl.ds(c * 16, 16)
                            acc_v[off, sl] = jnp.maximum(acc_v[off, sl],
                                                         rows_v[r, sl])

                    return 0

                lax.fori_loop(0, G, rbody, 0, unroll=False)

            @pl.when(ng > 0)
            def _():
                stage_and_start(0, idxa_v, rowsa_v, sema)

            npair = (ng + 1) // 2

            def pbody(p, _):
                e = 2 * p
                o = e + 1

                @pl.when(o < ng)
                def _():
                    stage_and_start(o, idxb_v, rowsb_v, semb)

                pltpu.make_async_copy(m_hbm.at[idxa_v], rowsa_v, sema).wait()
                accum(e, rowsa_v)

                @pl.when(o + 1 < ng)
                def _():
                    stage_and_start(o + 1, idxa_v, rowsa_v, sema)

                @pl.when(o < ng)
                def _():
                    pltpu.make_async_copy(m_hbm.at[idxb_v], rowsb_v, semb).wait()
                    accum(o, rowsb_v)

                return 0

            lax.fori_loop(0, npair, pbody, 0, unroll=False)

            pltpu.sync_copy(acc_v, agg_hbms[j].at[pl.ds(lo, _ROWS)])

    return k(*ms, *hits, *cnts)


def _mm(x, w):
    def body(x_ref, w_ref, o_ref):
        o_ref[...] = jnp.dot(x_ref[...], w_ref[...], preferred_element_type=f32)

    return pl.pallas_call(
        body,
        out_shape=jax.ShapeDtypeStruct((x.shape[0], w.shape[1]), f32),
    )(x, w)


def _edge_init(attr, w1, b1, w2, b2):
    R = 2048
    grid = E_PAD // R

    def body(a_ref, w1_ref, b1_ref, w2_ref, b2_ref, o_ref):
        h = jnp.maximum(
            jnp.dot(a_ref[...], w1_ref[...], preferred_element_type=f32)
            + b1_ref[...], 0.0)
        o_ref[...] = jnp.dot(h, w2_ref[...], preferred_element_type=f32) + b2_ref[...]

    full = lambda s: pl.BlockSpec(s, lambda i: (0, 0))
    return pl.pallas_call(
        body,
        grid=(grid,),
        in_specs=[
            pl.BlockSpec((R, 16), lambda i: (i, 0)),
            full((16, HH)), full((1, HH)), full((HH, HH)), full((1, HH)),
        ],
        out_specs=pl.BlockSpec((R, HH), lambda i: (i, 0)),
        out_shape=jax.ShapeDtypeStruct((E_PAD, HH), f32),
    )(attr, w1, b1.reshape(1, HH), w2, b2.reshape(1, HH))


def _edge_mlp(ea, g0, u0, g1, u1, w1e, b1, w2, b2, wf1, bf1, wf2, bf2):
    R = 2048
    grid = E_PAD // R
    k0 = g0.shape[1]
    k1 = g1.shape[1]

    def body(ea_ref, g0_ref, g1_ref, *refs):
        i = 0
        if u0 is not None:
            u0_ref = refs[i]; i += 1
        if u1 is not None:
            u1_ref = refs[i]; i += 1
        (w1e_ref, b1_ref, w2_ref, b2_ref,
         wf1_ref, bf1_ref, wf2_ref, bf2_ref, ean_ref, m_ref) = refs[i:]
        a = ea_ref[...]
        p0 = (jnp.dot(g0_ref[...], u0_ref[...], preferred_element_type=f32)
              if u0 is not None else g0_ref[...])
        p1 = (jnp.dot(g1_ref[...], u1_ref[...], preferred_element_type=f32)
              if u1 is not None else g1_ref[...])
        pre = (jnp.dot(a, w1e_ref[...], preferred_element_type=f32)
               + p0 + p1 + b1_ref[...])
        h = jnp.maximum(pre, 0.0)
        ean = a + jnp.dot(h, w2_ref[...], preferred_element_type=f32) + b2_ref[...]
        ean_ref[...] = ean
        h2 = jnp.maximum(
            jnp.dot(ean, wf1_ref[...], preferred_element_type=f32) + bf1_ref[...], 0.0)
        m_ref[...] = jnp.dot(h2, wf2_ref[...], preferred_element_type=f32) + bf2_ref[...]

    eb = pl.BlockSpec((R, HH), lambda i: (i, 0))
    full = lambda s: pl.BlockSpec(s, lambda i: (0, 0))
    ins = [ea, g0, g1]
    specs = [eb, pl.BlockSpec((R, k0), lambda i: (i, 0)),
             pl.BlockSpec((R, k1), lambda i: (i, 0))]
    if u0 is not None:
        ins.append(u0); specs.append(full((k0, HH)))
    if u1 is not None:
        ins.append(u1); specs.append(full((k1, HH)))
    ins += [w1e, b1.reshape(1, HH), w2, b2.reshape(1, HH),
            wf1, bf1.reshape(1, HH), wf2, bf2.reshape(1, HH)]
    specs += [full((HH, HH)), full((1, HH)), full((HH, HH)), full((1, HH)),
              full((HH, HH)), full((1, HH)), full((HH, HH)), full((1, HH))]
    return pl.pallas_call(
        body,
        grid=(grid,),
        in_specs=specs,
        out_specs=[eb, eb],
        out_shape=[jax.ShapeDtypeStruct((E_PAD, HH), f32),
                   jax.ShapeDtypeStruct((E_PAD, HH), f32)],
    )(*ins)


def _agent_update(xa, a0, a1, a2):
    def body(x_ref, a0_ref, a1_ref, a2_ref, o_ref):
        def fix(v):
            return jnp.where(jnp.isfinite(v), v, 0.0)
        o_ref[...] = x_ref[...] + jnp.maximum(
            jnp.maximum(fix(a0_ref[...]), fix(a1_ref[...])), fix(a2_ref[...]))

    return pl.pallas_call(
        body,
        out_shape=jax.ShapeDtypeStruct((A_PAD, HH), f32),
    )(xa, a0, a1, a2)


def _field(xa, act, w1v, w1a, b1, w2, b2):
    def body(x_ref, a_ref, w1v_ref, w1a_ref, b1_ref, w2_ref, b2_ref, o_ref):
        h = jnp.maximum(
            jnp.dot(x_ref[...], w1v_ref[...], preferred_element_type=f32)
            + jnp.dot(a_ref[...], w1a_ref[...], preferred_element_type=f32)
            + b1_ref[...], 0.0)
        o_ref[...] = jnp.dot(h, w2_ref[...], preferred_element_type=f32) + b2_ref[...]

    return pl.pallas_call(
        body,
        out_shape=jax.ShapeDtypeStruct((A_PAD, 1), f32),
    )(xa, act, w1v, w1a, b1.reshape(1, HH), w2, b2.reshape(1, 1))


def kernel(x_obstacle, x_agent, x_goal, action,
           edge_index_oa, edge_attr_oa,
           edge_index_aa, edge_attr_aa,
           edge_index_ga, edge_attr_ga, params):
    types = ["oa", "aa", "ga"]
    ei = {"oa": edge_index_oa, "aa": edge_index_aa, "ga": edge_index_ga}
    eattr = {"oa": edge_attr_oa, "aa": edge_attr_aa, "ga": edge_attr_ga}

    xo_raw = _padr(x_obstacle, O_PAD)
    xa_raw = _padr(x_agent, A_PAD)
    xg_raw = _padr(x_goal, G_PAD)
    W_emb = params["W_embed"]
    xo = _mm(xo_raw, W_emb)
    xa = _mm(xa_raw, W_emb)
    xg = _mm(xg_raw, W_emb)
    act = _padr(action, A_PAD)

    ea, idx0, idx1 = {}, {}, {}
    for t in types:
        pp = params["ee_" + t]
        ea[t] = _edge_init(_padr(eattr[t], E_PAD), pp["W1"], pp["b1"], pp["W2"], pp["b2"])
        e = ei[t]
        idx0[t] = jnp.pad(e[0], (0, E_PAD - e.shape[1]), constant_values=0).astype(i32)
        idx1[t] = jnp.pad(e[1], (0, E_PAD - e.shape[1]), constant_values=0).astype(i32)

    dsts = [jnp.pad(ei[t][1], (0, E_PAD - ei[t].shape[1]),
                    constant_values=SENTINEL).astype(i32) for t in types]
    hits_l, cnts_l = _sc_hits3(dsts)

    r6 = _sc_gather_multi(
        [xo_raw, xa_raw, xg_raw, xa_raw, xa_raw, xa_raw],
        [idx0["oa"], idx0["aa"], idx0["ga"],
         idx1["oa"], idx1["aa"], idx1["ga"]])
    rx0 = dict(zip(types, r6[:3]))
    rx1 = dict(zip(types, r6[3:]))

    def w1split(l, t):
        W1 = params["em_%d_%s" % (l, t)]["W1"]
        return W1[:HH], W1[HH:2 * HH], W1[2 * HH:]

    for l in range(2):
        gathered = {}
        if l == 1:
            _, W1s_aa, _ = w1split(1, "aa")
            projs = [_mm(xa, W1s_aa)] + [_mm(xa, w1split(1, t)[2])
                                         for t in types]
            g4 = _sc_gather_multi(
                projs, [idx0["aa"], idx1["oa"], idx1["aa"], idx1["ga"]])
            gathered = {"aa_src": g4[0], "oa_dst": g4[1],
                        "aa_dst": g4[2], "ga_dst": g4[3]}
        ms = []
        for t in types:
            em = params["em_%d_%s" % (l, t)]
            W1e, W1s, W1d = w1split(l, t)
            if l == 0:
                g0, u0 = rx0[t], _mm(W_emb, W1s)
                g1, u1 = rx1[t], _mm(W_emb, W1d)
            else:
                if t == "aa":
                    g0, u0 = gathered["aa_src"], None
                else:
                    g0, u0 = rx0[t], _mm(W_emb, W1s)
                g1, u1 = gathered[t + "_dst"], None
            fx = params["fx_%d_%s" % (l, t)]
            ea[t], m = _edge_mlp(ea[t], g0, u0, g1, u1,
                                 W1e, em["b1"], em["W2"], em["b2"],
                                 fx["W1"], fx["b1"], fx["W2"], fx["b2"])
            ms.append(m)
        aggs = _sc_segmax3(ms, hits_l, cnts_l)
        xa = _agent_update(xa, aggs[0], aggs[1], aggs[2])

    fp = params["field"]
    W1f = fp["W1"]
    out = _field(xa, act, W1f[:HH], W1f[HH:], fp["b1"], fp["W2"], fp["b2"])
    return out[:x_agent.shape[0], 0]


# speedup vs baseline: 1.5722x; 1.1633x over previous
import functools

import jax
import jax.numpy as jnp
from jax import lax
from jax.experimental import pallas as pl
from jax.experimental.pallas import tpu as pltpu
from jax.experimental.pallas import tpu_sc as plsc

HH = 256
E_PAD = 53248
A_PAD = 5120
O_PAD = 4096
G_PAD = 1024
NW = 32
SENTINEL = 1 << 20

f32 = jnp.float32
i32 = jnp.int32


def _padr(x, n, val=0.0):
    pads = ((0, n - x.shape[0]),) + ((0, 0),) * (x.ndim - 1)
    return jnp.pad(x, pads, constant_values=val)


def _sc_mesh():
    return plsc.VectorSubcoreMesh(core_axis_name="c", subcore_axis_name="s")


_SC_PARAMS = pltpu.CompilerParams(needs_layout_passes=False)


def _sc_gather_multi(tables, idxs, staged=None):
    W = tables[0].shape[1]
    assert all(t.shape[1] == W for t in tables)
    nj = len(tables)
    per_w = E_PAD // NW
    C = 128 if W <= 128 else 64
    nch = per_w // C
    NB = 4
    D = 2
    stage_shape = tables[staged[0]].shape if staged is not None else None

    @functools.partial(
        pl.kernel,
        mesh=_sc_mesh(),
        out_type=[jax.ShapeDtypeStruct((E_PAD, W), f32)] * nj,
        compiler_params=_SC_PARAMS,
        scratch_types=[pltpu.VMEM((per_w,), i32)]
        + [pltpu.VMEM((C, W), f32)] * NB
        + [pltpu.SemaphoreType.DMA] * NB
        + [pltpu.SemaphoreType.DMA] * NB
        + ([pltpu.VMEM_SHARED(stage_shape, f32)] if staged is not None else []),
    )
    def k(*refs):
        tabs = list(refs[:nj])
        idxr = refs[nj:2 * nj]
        outs = refs[2 * nj:3 * nj]
        idx_v = refs[3 * nj]
        rows = refs[3 * nj + 1:3 * nj + 1 + NB]
        gsem = refs[3 * nj + 1 + NB:3 * nj + 1 + 2 * NB]
        wsem = refs[3 * nj + 1 + 2 * NB:3 * nj + 1 + 3 * NB]
        wid = lax.axis_index("s") * 2 + lax.axis_index("c")
        base0 = wid * per_w

        if staged is not None:
            stab = refs[3 * nj + 1 + 3 * NB]
            sid = lax.axis_index("s")
            nrow = stage_shape[0] // 16
            pltpu.sync_copy(tabs[staged[0]].at[pl.ds(sid * nrow, nrow)],
                            stab.at[pl.ds(sid * nrow, nrow)])
            plsc.subcore_barrier()
            for j in staged:
                tabs[j] = stab

        for j in range(nj):
            pltpu.sync_copy(idxr[j].at[pl.ds(base0, per_w)], idx_v)
            gathers = [None] * nch
            writes = [None] * nch

            def start_gather(kk):
                b = kk % NB
                if kk >= NB:
                    writes[kk - NB].wait()
                gathers[kk] = pltpu.async_copy(
                    tabs[j].at[idx_v.at[pl.ds(kk * C, C)]], rows[b], gsem[b])

            for kk in range(min(D, nch)):
                start_gather(kk)
            for kk in range(nch):
                if kk + D < nch:
                    start_gather(kk + D)
                gathers[kk].wait()
                writes[kk] = pltpu.async_copy(
                    rows[kk % NB],
                    outs[j].at[pl.ds(base0 + kk * C, C)],
                    wsem[kk % NB])
            for kk in range(max(0, nch - NB), nch):
                writes[kk].wait()

    return k(*tables, *idxs)


_ROWS = A_PAD // NW
_HCH = 4096


def _sc_hits3(dsts):
    DCH = 2048
    nch = E_PAD // DCH
    nj = len(dsts)

    @functools.partial(
        pl.kernel,
        mesh=_sc_mesh(),
        out_type=[jax.ShapeDtypeStruct((NW, E_PAD), i32)] * nj
        + [jax.ShapeDtypeStruct((NW, 16), i32)] * nj,
        compiler_params=_SC_PARAMS,
        scratch_types=[
            pltpu.VMEM((DCH,), i32),
            pltpu.VMEM((E_PAD + 16,), i32),
            pltpu.VMEM((16,), i32),
        ],
    )
    def k(*refs):
        dst_hbms = refs[:nj]
        hits_hbms = refs[nj:2 * nj]
        cnt_hbms = refs[2 * nj:3 * nj]
        dbuf_v, hits_v, cnt_v = refs[3 * nj:]
        wid = lax.axis_index("s") * 2 + lax.axis_index("c")
        lo = wid * _ROWS
        hi = lo + _ROWS
        lane = lax.broadcasted_iota(i32, (16,), 0)

        for j in range(nj):
            def chunk_body(kk, cnt, dst_hbm=dst_hbms[j]):
                pltpu.sync_copy(dst_hbm.at[pl.ds(kk * DCH, DCH)], dbuf_v)

                def vbody(v, cnt):
                    d = dbuf_v[pl.ds(v * 16, 16)]
                    msk = (d >= lo) & (d < hi)
                    eid = kk * DCH + v * 16 + lane
                    packed = ((d - lo) << 16) | eid
                    pos = plsc.cumsum(msk.astype(i32))
                    plsc.store_scatter(hits_v, [cnt + pos - 1], packed, mask=msk)
                    return cnt + pos[15]

                return lax.fori_loop(0, DCH // 16, vbody, cnt, unroll=False)

            cnt = lax.fori_loop(0, nch, chunk_body, 0, unroll=False)

            cnt_v[...] = jnp.zeros((16,), i32) + cnt
            pltpu.sync_copy(cnt_v, cnt_hbms[j].at[wid])
            for c in range(E_PAD // _HCH):
                @pl.when(c * _HCH < cnt)
                def _(c=c, j=j):
                    pltpu.sync_copy(hits_v.at[pl.ds(c * _HCH, _HCH)],
                                    hits_hbms[j].at[wid, pl.ds(c * _HCH, _HCH)])

    outs = k(*dsts)
    return outs[:nj], outs[nj:]


def _sc_segmax3(ms, hits, cnts):
    G = 64
    nj = len(ms)

    @functools.partial(
        pl.kernel,
        mesh=_sc_mesh(),
        out_type=[jax.ShapeDtypeStruct((A_PAD, HH), f32)] * nj,
        compiler_params=_SC_PARAMS,
        scratch_types=[
            pltpu.VMEM((_ROWS, HH), f32),
            pltpu.VMEM((E_PAD + 16,), i32),
            pltpu.VMEM((16,), i32),
            pltpu.VMEM((G,), i32),
            pltpu.VMEM((G,), i32),
            pltpu.VMEM((G, HH), f32),
            pltpu.VMEM((G, HH), f32),
            pltpu.SemaphoreType.DMA,
            pltpu.SemaphoreType.DMA,
        ],
    )
    def k(*refs):
        m_hbms = refs[:nj]
        hits_hbms = refs[nj:2 * nj]
        cnt_hbms = refs[2 * nj:3 * nj]
        agg_hbms = refs[3 * nj:4 * nj]
        (acc_v, hits_v, cnt_v, idxa_v, idxb_v,
         rowsa_v, rowsb_v, sema, semb) = refs[4 * nj:]
        wid = lax.axis_index("s") * 2 + lax.axis_index("c")
        lo = wid * _ROWS
        lane = lax.broadcasted_iota(i32, (16,), 0)
        neginf = jnp.full((16,), -jnp.inf, f32)

        for j in range(nj):
            m_hbm = m_hbms[j]
            pltpu.sync_copy(cnt_hbms[j].at[wid], cnt_v)
            cnt = cnt_v[...][0]
            for c in range(E_PAD // _HCH):
                @pl.when(c * _HCH < cnt)
                def _(c=c, j=j):
                    pltpu.sync_copy(hits_hbms[j].at[wid, pl.ds(c * _HCH, _HCH)],
                                    hits_v.at[pl.ds(c * _HCH, _HCH)])

            def init_row(r, _):
                for c in range(HH // 16):
                    acc_v[r, pl.ds(c * 16, 16)] = neginf
                return 0

            lax.fori_loop(0, _ROWS, init_row, 0, unroll=False)

            ng = (cnt + G - 1) // G

            def stage_and_start(g, idxt_v, rows_v, sem):
                base = g * G
                for vv in range(G // 16):
                    pos = base + vv * 16
                    p = hits_v[pl.ds(pos, 16)]
                    valid = (pos + lane) < cnt
                    idxt_v[pl.ds(vv * 16, 16)] = jnp.where(valid, p & 0xFFFF, 0)
                return pltpu.async_copy(m_hbm.at[idxt_v], rows_v, sem)

            def accum(g, rows_v):
                base = g * G
                count = jnp.minimum(G, cnt - base)

                def rbody(r, _):
                    pk = hits_v[pl.ds(base + r, 16)][0]
                    off = pk >> 16

                    @pl.when(r < count)
                    def _():
                        for c in range(HH // 16):
                            sl = pl.ds(c * 16, 16)
                            acc_v[off, sl] = jnp.maximum(acc_v[off, sl],
                                                         rows_v[r, sl])

                    return 0

                lax.fori_loop(0, G, rbody, 0, unroll=False)

            @pl.when(ng > 0)
            def _():
                stage_and_start(0, idxa_v, rowsa_v, sema)

            npair = (ng + 1) // 2

            def pbody(p, _):
                e = 2 * p
                o = e + 1

                @pl.when(o < ng)
                def _():
                    stage_and_start(o, idxb_v, rowsb_v, semb)

                pltpu.make_async_copy(m_hbm.at[idxa_v], rowsa_v, sema).wait()
                accum(e, rowsa_v)

                @pl.when(o + 1 < ng)
                def _():
                    stage_and_start(o + 1, idxa_v, rowsa_v, sema)

                @pl.when(o < ng)
                def _():
                    pltpu.make_async_copy(m_hbm.at[idxb_v], rowsb_v, semb).wait()
                    accum(o, rowsb_v)

                return 0

            lax.fori_loop(0, npair, pbody, 0, unroll=False)

            pltpu.sync_copy(acc_v, agg_hbms[j].at[pl.ds(lo, _ROWS)])

    return k(*ms, *hits, *cnts)


def _mm(x, w):
    def body(x_ref, w_ref, o_ref):
        o_ref[...] = jnp.dot(x_ref[...], w_ref[...], preferred_element_type=f32)

    return pl.pallas_call(
        body,
        out_shape=jax.ShapeDtypeStruct((x.shape[0], w.shape[1]), f32),
    )(x, w)


def _edge_init(attr, w1, b1, w2, b2):
    R = 2048
    grid = E_PAD // R

    def body(a_ref, w1_ref, b1_ref, w2_ref, b2_ref, o_ref):
        h = jnp.maximum(
            jnp.dot(a_ref[...], w1_ref[...], preferred_element_type=f32)
            + b1_ref[...], 0.0)
        o_ref[...] = jnp.dot(h, w2_ref[...], preferred_element_type=f32) + b2_ref[...]

    full = lambda s: pl.BlockSpec(s, lambda i: (0, 0))
    return pl.pallas_call(
        body,
        grid=(grid,),
        in_specs=[
            pl.BlockSpec((R, 16), lambda i: (i, 0)),
            full((16, HH)), full((1, HH)), full((HH, HH)), full((1, HH)),
        ],
        out_specs=pl.BlockSpec((R, HH), lambda i: (i, 0)),
        out_shape=jax.ShapeDtypeStruct((E_PAD, HH), f32),
    )(attr, w1, b1.reshape(1, HH), w2, b2.reshape(1, HH))


def _edge_mlp(ea, g0, u0, g1, u1, w1e, b1, w2, b2, wf1, bf1, wf2, bf2):
    R = 2048
    grid = E_PAD // R
    k0 = g0.shape[1]
    k1 = g1.shape[1]

    def body(ea_ref, g0_ref, g1_ref, *refs):
        i = 0
        if u0 is not None:
            u0_ref = refs[i]; i += 1
        if u1 is not None:
            u1_ref = refs[i]; i += 1
        (w1e_ref, b1_ref, w2_ref, b2_ref,
         wf1_ref, bf1_ref, wf2_ref, bf2_ref, ean_ref, m_ref) = refs[i:]
        a = ea_ref[...]
        p0 = (jnp.dot(g0_ref[...], u0_ref[...], preferred_element_type=f32)
              if u0 is not None else g0_ref[...])
        p1 = (jnp.dot(g1_ref[...], u1_ref[...], preferred_element_type=f32)
              if u1 is not None else g1_ref[...])
        pre = (jnp.dot(a, w1e_ref[...], preferred_element_type=f32)
               + p0 + p1 + b1_ref[...])
        h = jnp.maximum(pre, 0.0)
        ean = a + jnp.dot(h, w2_ref[...], preferred_element_type=f32) + b2_ref[...]
        ean_ref[...] = ean
        h2 = jnp.maximum(
            jnp.dot(ean, wf1_ref[...], preferred_element_type=f32) + bf1_ref[...], 0.0)
        m_ref[...] = jnp.dot(h2, wf2_ref[...], preferred_element_type=f32) + bf2_ref[...]

    eb = pl.BlockSpec((R, HH), lambda i: (i, 0))
    full = lambda s: pl.BlockSpec(s, lambda i: (0, 0))
    ins = [ea, g0, g1]
    specs = [eb, pl.BlockSpec((R, k0), lambda i: (i, 0)),
             pl.BlockSpec((R, k1), lambda i: (i, 0))]
    if u0 is not None:
        ins.append(u0); specs.append(full((k0, HH)))
    if u1 is not None:
        ins.append(u1); specs.append(full((k1, HH)))
    ins += [w1e, b1.reshape(1, HH), w2, b2.reshape(1, HH),
            wf1, bf1.reshape(1, HH), wf2, bf2.reshape(1, HH)]
    specs += [full((HH, HH)), full((1, HH)), full((HH, HH)), full((1, HH)),
              full((HH, HH)), full((1, HH)), full((HH, HH)), full((1, HH))]
    return pl.pallas_call(
        body,
        grid=(grid,),
        in_specs=specs,
        out_specs=[eb, eb],
        out_shape=[jax.ShapeDtypeStruct((E_PAD, HH), f32),
                   jax.ShapeDtypeStruct((E_PAD, HH), f32)],
    )(*ins)


def _agent_update(xa, a0, a1, a2):
    def body(x_ref, a0_ref, a1_ref, a2_ref, o_ref):
        def fix(v):
            return jnp.where(jnp.isfinite(v), v, 0.0)
        o_ref[...] = x_ref[...] + jnp.maximum(
            jnp.maximum(fix(a0_ref[...]), fix(a1_ref[...])), fix(a2_ref[...]))

    return pl.pallas_call(
        body,
        out_shape=jax.ShapeDtypeStruct((A_PAD, HH), f32),
    )(xa, a0, a1, a2)


def _field(xa, act, w1v, w1a, b1, w2, b2):
    def body(x_ref, a_ref, w1v_ref, w1a_ref, b1_ref, w2_ref, b2_ref, o_ref):
        h = jnp.maximum(
            jnp.dot(x_ref[...], w1v_ref[...], preferred_element_type=f32)
            + jnp.dot(a_ref[...], w1a_ref[...], preferred_element_type=f32)
            + b1_ref[...], 0.0)
        o_ref[...] = jnp.dot(h, w2_ref[...], preferred_element_type=f32) + b2_ref[...]

    return pl.pallas_call(
        body,
        out_shape=jax.ShapeDtypeStruct((A_PAD, 1), f32),
    )(xa, act, w1v, w1a, b1.reshape(1, HH), w2, b2.reshape(1, 1))


def kernel(x_obstacle, x_agent, x_goal, action,
           edge_index_oa, edge_attr_oa,
           edge_index_aa, edge_attr_aa,
           edge_index_ga, edge_attr_ga, params):
    types = ["oa", "aa", "ga"]
    ei = {"oa": edge_index_oa, "aa": edge_index_aa, "ga": edge_index_ga}
    eattr = {"oa": edge_attr_oa, "aa": edge_attr_aa, "ga": edge_attr_ga}

    xo_raw = _padr(x_obstacle, O_PAD)
    xa_raw = _padr(x_agent, A_PAD)
    xg_raw = _padr(x_goal, G_PAD)
    W_emb = params["W_embed"]
    xo = _mm(xo_raw, W_emb)
    xa = _mm(xa_raw, W_emb)
    xg = _mm(xg_raw, W_emb)
    act = _padr(action, A_PAD)

    ea, idx0, idx1 = {}, {}, {}
    for t in types:
        pp = params["ee_" + t]
        ea[t] = _edge_init(_padr(eattr[t], E_PAD), pp["W1"], pp["b1"], pp["W2"], pp["b2"])
        e = ei[t]
        idx0[t] = jnp.pad(e[0], (0, E_PAD - e.shape[1]), constant_values=0).astype(i32)
        idx1[t] = jnp.pad(e[1], (0, E_PAD - e.shape[1]), constant_values=0).astype(i32)

    dsts = [jnp.pad(ei[t][1], (0, E_PAD - ei[t].shape[1]),
                    constant_values=SENTINEL).astype(i32) for t in types]
    hits_l, cnts_l = _sc_hits3(dsts)

    r6 = _sc_gather_multi(
        [xo_raw, xa_raw, xg_raw, xa_raw, xa_raw, xa_raw],
        [idx0["oa"], idx0["aa"], idx0["ga"],
         idx1["oa"], idx1["aa"], idx1["ga"]],
        staged=[1, 3, 4, 5])
    rx0 = dict(zip(types, r6[:3]))
    rx1 = dict(zip(types, r6[3:]))

    def w1split(l, t):
        W1 = params["em_%d_%s" % (l, t)]["W1"]
        return W1[:HH], W1[HH:2 * HH], W1[2 * HH:]

    for l in range(2):
        gathered = {}
        if l == 1:
            _, W1s_aa, _ = w1split(1, "aa")
            projs = [_mm(xa, W1s_aa)] + [_mm(xa, w1split(1, t)[2])
                                         for t in types]
            g4 = _sc_gather_multi(
                projs, [idx0["aa"], idx1["oa"], idx1["aa"], idx1["ga"]])
            gathered = {"aa_src": g4[0], "oa_dst": g4[1],
                        "aa_dst": g4[2], "ga_dst": g4[3]}
        ms = []
        for t in types:
            em = params["em_%d_%s" % (l, t)]
            W1e, W1s, W1d = w1split(l, t)
            if l == 0:
                g0, u0 = rx0[t], _mm(W_emb, W1s)
                g1, u1 = rx1[t], _mm(W_emb, W1d)
            else:
                if t == "aa":
                    g0, u0 = gathered["aa_src"], None
                else:
                    g0, u0 = rx0[t], _mm(W_emb, W1s)
                g1, u1 = gathered[t + "_dst"], None
            fx = params["fx_%d_%s" % (l, t)]
            ea[t], m = _edge_mlp(ea[t], g0, u0, g1, u1,
                                 W1e, em["b1"], em["W2"], em["b2"],
                                 fx["W1"], fx["b1"], fx["W2"], fx["b2"])
            ms.append(m)
        aggs = _sc_segmax3(ms, hits_l, cnts_l)
        xa = _agent_update(xa, aggs[0], aggs[1], aggs[2])

    fp = params["field"]
    W1f = fp["W1"]
    out = _field(xa, act, W1f[:HH], W1f[HH:], fp["b1"], fp["W2"], fp["b2"])
    return out[:x_agent.shape[0], 0]


# speedup vs baseline: 2.2618x; 1.4387x over previous
import functools

import jax
import jax.numpy as jnp
from jax import lax
from jax.experimental import pallas as pl
from jax.experimental.pallas import tpu as pltpu
from jax.experimental.pallas import tpu_sc as plsc

HH = 256
E_PAD = 53248
A_PAD = 5120
O_PAD = 4096
G_PAD = 1024
NW = 32
SENTINEL = 1 << 20

f32 = jnp.float32
i32 = jnp.int32


def _padr(x, n, val=0.0):
    pads = ((0, n - x.shape[0]),) + ((0, 0),) * (x.ndim - 1)
    return jnp.pad(x, pads, constant_values=val)


def _sc_mesh():
    return plsc.VectorSubcoreMesh(core_axis_name="c", subcore_axis_name="s")


_SC_PARAMS = pltpu.CompilerParams(needs_layout_passes=False)


def _sc_gather_multi(tables, idxs, staged=None):
    W = tables[0].shape[1]
    assert all(t.shape[1] == W for t in tables)
    nj = len(tables)
    per_w = E_PAD // NW
    C = 64 if W <= 128 else 32
    nch = per_w // C
    NB = 3
    D = 2
    staged = staged or {}
    stage_pos = sorted(staged)

    @functools.partial(
        pl.kernel,
        mesh=_sc_mesh(),
        out_type=[jax.ShapeDtypeStruct((E_PAD, W), f32)] * nj,
        compiler_params=_SC_PARAMS,
        scratch_types=[pltpu.VMEM((per_w,), i32)]
        + [pltpu.VMEM((C, W), f32)] * NB
        + [pltpu.SemaphoreType.DMA] * NB
        + [pltpu.SemaphoreType.DMA] * NB
        + [pltpu.VMEM_SHARED(tables[p].shape, f32) for p in stage_pos],
    )
    def k(*refs):
        tabs = list(refs[:nj])
        idxr = refs[nj:2 * nj]
        outs = refs[2 * nj:3 * nj]
        idx_v = refs[3 * nj]
        rows = refs[3 * nj + 1:3 * nj + 1 + NB]
        gsem = refs[3 * nj + 1 + NB:3 * nj + 1 + 2 * NB]
        wsem = refs[3 * nj + 1 + 2 * NB:3 * nj + 1 + 3 * NB]
        stabs = refs[3 * nj + 1 + 3 * NB:]
        wid = lax.axis_index("s") * 2 + lax.axis_index("c")
        base0 = wid * per_w

        if stage_pos:
            sid = lax.axis_index("s")
            for si, p in enumerate(stage_pos):
                nrow = tables[p].shape[0] // 16
                pltpu.sync_copy(tabs[p].at[pl.ds(sid * nrow, nrow)],
                                stabs[si].at[pl.ds(sid * nrow, nrow)])
            plsc.subcore_barrier()
            for si, p in enumerate(stage_pos):
                for j in staged[p]:
                    tabs[j] = stabs[si]

        for j in range(nj):
            pltpu.sync_copy(idxr[j].at[pl.ds(base0, per_w)], idx_v)
            gathers = [None] * nch
            writes = [None] * nch

            def start_gather(kk):
                b = kk % NB
                if kk >= NB:
                    writes[kk - NB].wait()
                gathers[kk] = pltpu.async_copy(
                    tabs[j].at[idx_v.at[pl.ds(kk * C, C)]], rows[b], gsem[b])

            for kk in range(min(D, nch)):
                start_gather(kk)
            for kk in range(nch):
                if kk + D < nch:
                    start_gather(kk + D)
                gathers[kk].wait()
                writes[kk] = pltpu.async_copy(
                    rows[kk % NB],
                    outs[j].at[pl.ds(base0 + kk * C, C)],
                    wsem[kk % NB])
            for kk in range(max(0, nch - NB), nch):
                writes[kk].wait()

    return k(*tables, *idxs)


_ROWS = A_PAD // NW
_HCH = 4096


def _sc_hits3(dsts):
    DCH = 2048
    nch = E_PAD // DCH
    nj = len(dsts)

    @functools.partial(
        pl.kernel,
        mesh=_sc_mesh(),
        out_type=[jax.ShapeDtypeStruct((NW, E_PAD), i32)] * nj
        + [jax.ShapeDtypeStruct((NW, 16), i32)] * nj,
        compiler_params=_SC_PARAMS,
        scratch_types=[
            pltpu.VMEM((DCH,), i32),
            pltpu.VMEM((E_PAD + 16,), i32),
            pltpu.VMEM((16,), i32),
        ],
    )
    def k(*refs):
        dst_hbms = refs[:nj]
        hits_hbms = refs[nj:2 * nj]
        cnt_hbms = refs[2 * nj:3 * nj]
        dbuf_v, hits_v, cnt_v = refs[3 * nj:]
        wid = lax.axis_index("s") * 2 + lax.axis_index("c")
        lo = wid * _ROWS
        hi = lo + _ROWS
        lane = lax.broadcasted_iota(i32, (16,), 0)

        for j in range(nj):
            def chunk_body(kk, cnt, dst_hbm=dst_hbms[j]):
                pltpu.sync_copy(dst_hbm.at[pl.ds(kk * DCH, DCH)], dbuf_v)

                def vbody(v, cnt):
                    d = dbuf_v[pl.ds(v * 16, 16)]
                    msk = (d >= lo) & (d < hi)
                    eid = kk * DCH + v * 16 + lane
                    packed = ((d - lo) << 16) | eid
                    pos = plsc.cumsum(msk.astype(i32))
                    plsc.store_scatter(hits_v, [cnt + pos - 1], packed, mask=msk)
                    return cnt + pos[15]

                return lax.fori_loop(0, DCH // 16, vbody, cnt, unroll=False)

            cnt = lax.fori_loop(0, nch, chunk_body, 0, unroll=False)

            cnt_v[...] = jnp.zeros((16,), i32) + cnt
            pltpu.sync_copy(cnt_v, cnt_hbms[j].at[wid])
            for c in range(E_PAD // _HCH):
                @pl.when(c * _HCH < cnt)
                def _(c=c, j=j):
                    pltpu.sync_copy(hits_v.at[pl.ds(c * _HCH, _HCH)],
                                    hits_hbms[j].at[wid, pl.ds(c * _HCH, _HCH)])

    outs = k(*dsts)
    return outs[:nj], outs[nj:]


def _sc_segmax3(ms, hits, cnts):
    G = 64
    nj = len(ms)

    @functools.partial(
        pl.kernel,
        mesh=_sc_mesh(),
        out_type=[jax.ShapeDtypeStruct((A_PAD, HH), f32)] * nj,
        compiler_params=_SC_PARAMS,
        scratch_types=[
            pltpu.VMEM((_ROWS, HH), f32),
            pltpu.VMEM((E_PAD + 16,), i32),
            pltpu.VMEM((16,), i32),
            pltpu.VMEM((G,), i32),
            pltpu.VMEM((G,), i32),
            pltpu.VMEM((G, HH), f32),
            pltpu.VMEM((G, HH), f32),
            pltpu.SemaphoreType.DMA,
            pltpu.SemaphoreType.DMA,
        ],
    )
    def k(*refs):
        m_hbms = refs[:nj]
        hits_hbms = refs[nj:2 * nj]
        cnt_hbms = refs[2 * nj:3 * nj]
        agg_hbms = refs[3 * nj:4 * nj]
        (acc_v, hits_v, cnt_v, idxa_v, idxb_v,
         rowsa_v, rowsb_v, sema, semb) = refs[4 * nj:]
        wid = lax.axis_index("s") * 2 + lax.axis_index("c")
        lo = wid * _ROWS
        lane = lax.broadcasted_iota(i32, (16,), 0)
        neginf = jnp.full((16,), -jnp.inf, f32)

        for j in range(nj):
            m_hbm = m_hbms[j]
            pltpu.sync_copy(cnt_hbms[j].at[wid], cnt_v)
            cnt = cnt_v[...][0]
            for c in range(E_PAD // _HCH):
                @pl.when(c * _HCH < cnt)
                def _(c=c, j=j):
                    pltpu.sync_copy(hits_hbms[j].at[wid, pl.ds(c * _HCH, _HCH)],
                                    hits_v.at[pl.ds(c * _HCH, _HCH)])

            def init_row(r, _):
                for c in range(HH // 16):
                    acc_v[r, pl.ds(c * 16, 16)] = neginf
                return 0

            lax.fori_loop(0, _ROWS, init_row, 0, unroll=False)

            ng = (cnt + G - 1) // G

            def stage_and_start(g, idxt_v, rows_v, sem):
                base = g * G
                for vv in range(G // 16):
                    pos = base + vv * 16
                    p = hits_v[pl.ds(pos, 16)]
                    valid = (pos + lane) < cnt
                    idxt_v[pl.ds(vv * 16, 16)] = jnp.where(valid, p & 0xFFFF, 0)
                return pltpu.async_copy(m_hbm.at[idxt_v], rows_v, sem)

            def accum(g, rows_v):
                base = g * G
                count = jnp.minimum(G, cnt - base)

                def rbody(r, _):
                    pk = hits_v[pl.ds(base + r, 16)][0]
                    off = pk >> 16

                    @pl.when(r < count)
                    def _():
                        for c in range(HH // 16):
                            sl = pl.ds(c * 16, 16)
                            acc_v[off, sl] = jnp.maximum(acc_v[off, sl],
                                                         rows_v[r, sl])

                    return 0

                lax.fori_loop(0, G, rbody, 0, unroll=False)

            @pl.when(ng > 0)
            def _():
                stage_and_start(0, idxa_v, rowsa_v, sema)

            npair = (ng + 1) // 2

            def pbody(p, _):
                e = 2 * p
                o = e + 1

                @pl.when(o < ng)
                def _():
                    stage_and_start(o, idxb_v, rowsb_v, semb)

                pltpu.make_async_copy(m_hbm.at[idxa_v], rowsa_v, sema).wait()
                accum(e, rowsa_v)

                @pl.when(o + 1 < ng)
                def _():
                    stage_and_start(o + 1, idxa_v, rowsa_v, sema)

                @pl.when(o < ng)
                def _():
                    pltpu.make_async_copy(m_hbm.at[idxb_v], rowsb_v, semb).wait()
                    accum(o, rowsb_v)

                return 0

            lax.fori_loop(0, npair, pbody, 0, unroll=False)

            pltpu.sync_copy(acc_v, agg_hbms[j].at[pl.ds(lo, _ROWS)])

    return k(*ms, *hits, *cnts)


def _mm(x, w):
    def body(x_ref, w_ref, o_ref):
        o_ref[...] = jnp.dot(x_ref[...], w_ref[...], preferred_element_type=f32)

    return pl.pallas_call(
        body,
        out_shape=jax.ShapeDtypeStruct((x.shape[0], w.shape[1]), f32),
    )(x, w)


def _edge_init(attr, w1, b1, w2, b2):
    R = 2048
    grid = E_PAD // R

    def body(a_ref, w1_ref, b1_ref, w2_ref, b2_ref, o_ref):
        h = jnp.maximum(
            jnp.dot(a_ref[...], w1_ref[...], preferred_element_type=f32)
            + b1_ref[...], 0.0)
        o_ref[...] = jnp.dot(h, w2_ref[...], preferred_element_type=f32) + b2_ref[...]

    full = lambda s: pl.BlockSpec(s, lambda i: (0, 0))
    return pl.pallas_call(
        body,
        grid=(grid,),
        in_specs=[
            pl.BlockSpec((R, 16), lambda i: (i, 0)),
            full((16, HH)), full((1, HH)), full((HH, HH)), full((1, HH)),
        ],
        out_specs=pl.BlockSpec((R, HH), lambda i: (i, 0)),
        out_shape=jax.ShapeDtypeStruct((E_PAD, HH), f32),
    )(attr, w1, b1.reshape(1, HH), w2, b2.reshape(1, HH))


def _edge_mlp(ea, src0, src1, w1e, b1, w2, b2, wf1, bf1, wf2, bf2):
    R = 2048
    grid = E_PAD // R
    pairs = list(src0) + list(src1)

    def body(ea_ref, *refs):
        gr = refs[:len(pairs)]
        refs = refs[len(pairs):]
        ur = []
        i = 0
        for g, u in pairs:
            if u is not None:
                ur.append(refs[i]); i += 1
            else:
                ur.append(None)
        (w1e_ref, b1_ref, w2_ref, b2_ref,
         wf1_ref, bf1_ref, wf2_ref, bf2_ref, ean_ref, m_ref) = refs[i:]

        def proj(j):
            return (jnp.dot(gr[j][...], ur[j][...], preferred_element_type=f32)
                    if ur[j] is not None else gr[j][...])

        a = ea_ref[...]
        pre = jnp.dot(a, w1e_ref[...], preferred_element_type=f32) + b1_ref[...]
        for j in range(len(pairs)):
            pre = pre + proj(j)
        h = jnp.maximum(pre, 0.0)
        ean = a + jnp.dot(h, w2_ref[...], preferred_element_type=f32) + b2_ref[...]
        ean_ref[...] = ean
        h2 = jnp.maximum(
            jnp.dot(ean, wf1_ref[...], preferred_element_type=f32) + bf1_ref[...], 0.0)
        m_ref[...] = jnp.dot(h2, wf2_ref[...], preferred_element_type=f32) + bf2_ref[...]

    eb = pl.BlockSpec((R, HH), lambda i: (i, 0))
    full = lambda s: pl.BlockSpec(s, lambda i: (0, 0))
    ins = [ea]
    specs = [eb]
    for g, u in pairs:
        ins.append(g)
        kg = g.shape[1]
        specs.append(pl.BlockSpec((R, kg), lambda i: (i, 0)))
    for g, u in pairs:
        if u is not None:
            ins.append(u)
            specs.append(full(u.shape))
    ins += [w1e, b1.reshape(1, HH), w2, b2.reshape(1, HH),
            wf1, bf1.reshape(1, HH), wf2, bf2.reshape(1, HH)]
    specs += [full((HH, HH)), full((1, HH)), full((HH, HH)), full((1, HH)),
              full((HH, HH)), full((1, HH)), full((HH, HH)), full((1, HH))]
    return pl.pallas_call(
        body,
        grid=(grid,),
        in_specs=specs,
        out_specs=[eb, eb],
        out_shape=[jax.ShapeDtypeStruct((E_PAD, HH), f32),
                   jax.ShapeDtypeStruct((E_PAD, HH), f32)],
    )(*ins)


def _agent_update(xa, a0, a1, a2):
    def body(x_ref, a0_ref, a1_ref, a2_ref, o_ref):
        def fix(v):
            return jnp.where(jnp.isfinite(v), v, 0.0)
        o_ref[...] = x_ref[...] + jnp.maximum(
            jnp.maximum(fix(a0_ref[...]), fix(a1_ref[...])), fix(a2_ref[...]))

    return pl.pallas_call(
        body,
        out_shape=jax.ShapeDtypeStruct((A_PAD, HH), f32),
    )(xa, a0, a1, a2)


def _field(xa, act, w1v, w1a, b1, w2, b2):
    def body(x_ref, a_ref, w1v_ref, w1a_ref, b1_ref, w2_ref, b2_ref, o_ref):
        h = jnp.maximum(
            jnp.dot(x_ref[...], w1v_ref[...], preferred_element_type=f32)
            + jnp.dot(a_ref[...], w1a_ref[...], preferred_element_type=f32)
            + b1_ref[...], 0.0)
        o_ref[...] = jnp.dot(h, w2_ref[...], preferred_element_type=f32) + b2_ref[...]

    return pl.pallas_call(
        body,
        out_shape=jax.ShapeDtypeStruct((A_PAD, 1), f32),
    )(xa, act, w1v, w1a, b1.reshape(1, HH), w2, b2.reshape(1, 1))


def kernel(x_obstacle, x_agent, x_goal, action,
           edge_index_oa, edge_attr_oa,
           edge_index_aa, edge_attr_aa,
           edge_index_ga, edge_attr_ga, params):
    types = ["oa", "aa", "ga"]
    ei = {"oa": edge_index_oa, "aa": edge_index_aa, "ga": edge_index_ga}
    eattr = {"oa": edge_attr_oa, "aa": edge_attr_aa, "ga": edge_attr_ga}

    xo_raw = _padr(x_obstacle, O_PAD)
    xa_raw = _padr(x_agent, A_PAD)
    xg_raw = _padr(x_goal, G_PAD)
    W_emb = params["W_embed"]
    xo = _mm(xo_raw, W_emb)
    xa = _mm(xa_raw, W_emb)
    xg = _mm(xg_raw, W_emb)
    act = _padr(action, A_PAD)

    ea, idx0, idx1 = {}, {}, {}
    for t in types:
        pp = params["ee_" + t]
        ea[t] = _edge_init(_padr(eattr[t], E_PAD), pp["W1"], pp["b1"], pp["W2"], pp["b2"])
        e = ei[t]
        idx0[t] = jnp.pad(e[0], (0, E_PAD - e.shape[1]), constant_values=0).astype(i32)
        idx1[t] = jnp.pad(e[1], (0, E_PAD - e.shape[1]), constant_values=0).astype(i32)

    dsts = [jnp.pad(ei[t][1], (0, E_PAD - ei[t].shape[1]),
                    constant_values=SENTINEL).astype(i32) for t in types]
    hits_l, cnts_l = _sc_hits3(dsts)

    r6 = _sc_gather_multi(
        [xo_raw, xa_raw, xg_raw, xa_raw, xa_raw, xa_raw],
        [idx0["oa"], idx0["aa"], idx0["ga"],
         idx1["oa"], idx1["aa"], idx1["ga"]],
        staged={0: [0], 1: [1, 3, 4, 5], 2: [2]})
    rx0 = dict(zip(types, r6[:3]))
    rx1 = dict(zip(types, r6[3:]))

    def w1split(l, t):
        W1 = params["em_%d_%s" % (l, t)]["W1"]
        return W1[:HH], W1[HH:2 * HH], W1[2 * HH:]

    for l in range(2):
        gathered = {}
        if l == 1:
            xa_lo, xa_hi = xa[:, :128], xa[:, 128:]
            idx_l1 = [idx0["aa"], idx1["oa"], idx1["aa"], idx1["ga"]]
            g8 = _sc_gather_multi(
                [xa_lo, xa_hi] * 4,
                [ix for ix in idx_l1 for _ in range(2)],
                staged={0: [0, 2, 4, 6], 1: [1, 3, 5, 7]})
            halves = {"aa_src": g8[0:2], "oa_dst": g8[2:4],
                      "aa_dst": g8[4:6], "ga_dst": g8[6:8]}
        ms = []
        for t in types:
            em = params["em_%d_%s" % (l, t)]
            W1e, W1s, W1d = w1split(l, t)
            if l == 0:
                src0 = [(rx0[t], _mm(W_emb, W1s))]
                src1 = [(rx1[t], _mm(W_emb, W1d))]
            else:
                if t == "aa":
                    glo, ghi = halves["aa_src"]
                    src0 = [(glo, W1s[:128]), (ghi, W1s[128:])]
                else:
                    src0 = [(rx0[t], _mm(W_emb, W1s))]
                glo, ghi = halves[t + "_dst"]
                src1 = [(glo, W1d[:128]), (ghi, W1d[128:])]
            fx = params["fx_%d_%s" % (l, t)]
            ea[t], m = _edge_mlp(ea[t], src0, src1,
                                 W1e, em["b1"], em["W2"], em["b2"],
                                 fx["W1"], fx["b1"], fx["W2"], fx["b2"])
            ms.append(m)
        aggs = _sc_segmax3(ms, hits_l, cnts_l)
        xa = _agent_update(xa, aggs[0], aggs[1], aggs[2])

    fp = params["field"]
    W1f = fp["W1"]
    out = _field(xa, act, W1f[:HH], W1f[HH:], fp["b1"], fp["W2"], fp["b2"])
    return out[:x_agent.shape[0], 0]


# speedup vs baseline: 2.5673x; 1.1351x over previous
import functools

import jax
import jax.numpy as jnp
from jax import lax
from jax.experimental import pallas as pl
from jax.experimental.pallas import tpu as pltpu
from jax.experimental.pallas import tpu_sc as plsc

HH = 256
E_PAD = 53248
A_PAD = 5120
O_PAD = 4096
G_PAD = 1024
NW = 32
SENTINEL = 1 << 20

f32 = jnp.float32
i32 = jnp.int32


def _padr(x, n, val=0.0):
    pads = ((0, n - x.shape[0]),) + ((0, 0),) * (x.ndim - 1)
    return jnp.pad(x, pads, constant_values=val)


def _sc_mesh():
    return plsc.VectorSubcoreMesh(core_axis_name="c", subcore_axis_name="s")


_SC_PARAMS = pltpu.CompilerParams(needs_layout_passes=False)


def _sc_gather_multi(tables, idxs, staged=None):
    W = tables[0].shape[1]
    assert all(t.shape[1] == W for t in tables)
    nj = len(tables)
    per_w = E_PAD // NW
    C = 64 if W <= 128 else 32
    nch = per_w // C
    NB = 3
    D = 2
    staged = staged or {}
    stage_pos = sorted(staged)

    @functools.partial(
        pl.kernel,
        mesh=_sc_mesh(),
        out_type=[jax.ShapeDtypeStruct((E_PAD, W), f32)] * nj,
        compiler_params=_SC_PARAMS,
        scratch_types=[pltpu.VMEM((per_w,), i32)]
        + [pltpu.VMEM((C, W), f32)] * NB
        + [pltpu.SemaphoreType.DMA] * NB
        + [pltpu.SemaphoreType.DMA] * NB
        + [pltpu.VMEM_SHARED(tables[p].shape, f32) for p in stage_pos],
    )
    def k(*refs):
        tabs = list(refs[:nj])
        idxr = refs[nj:2 * nj]
        outs = refs[2 * nj:3 * nj]
        idx_v = refs[3 * nj]
        rows = refs[3 * nj + 1:3 * nj + 1 + NB]
        gsem = refs[3 * nj + 1 + NB:3 * nj + 1 + 2 * NB]
        wsem = refs[3 * nj + 1 + 2 * NB:3 * nj + 1 + 3 * NB]
        stabs = refs[3 * nj + 1 + 3 * NB:]
        wid = lax.axis_index("s") * 2 + lax.axis_index("c")
        base0 = wid * per_w

        if stage_pos:
            sid = lax.axis_index("s")
            for si, p in enumerate(stage_pos):
                nrow = tables[p].shape[0] // 16
                pltpu.sync_copy(tabs[p].at[pl.ds(sid * nrow, nrow)],
                                stabs[si].at[pl.ds(sid * nrow, nrow)])
            plsc.subcore_barrier()
            for si, p in enumerate(stage_pos):
                for j in staged[p]:
                    tabs[j] = stabs[si]

        for j in range(nj):
            pltpu.sync_copy(idxr[j].at[pl.ds(base0, per_w)], idx_v)
            gathers = [None] * nch
            writes = [None] * nch

            def start_gather(kk):
                b = kk % NB
                if kk >= NB:
                    writes[kk - NB].wait()
                gathers[kk] = pltpu.async_copy(
                    tabs[j].at[idx_v.at[pl.ds(kk * C, C)]], rows[b], gsem[b])

            for kk in range(min(D, nch)):
                start_gather(kk)
            for kk in range(nch):
                if kk + D < nch:
                    start_gather(kk + D)
                gathers[kk].wait()
                writes[kk] = pltpu.async_copy(
                    rows[kk % NB],
                    outs[j].at[pl.ds(base0 + kk * C, C)],
                    wsem[kk % NB])
            for kk in range(max(0, nch - NB), nch):
                writes[kk].wait()

    return k(*tables, *idxs)


_ROWS = A_PAD // NW
_HCH = 4096


def _sc_hits3(dsts):
    DCH = 2048
    nch = E_PAD // DCH
    nj = len(dsts)

    @functools.partial(
        pl.kernel,
        mesh=_sc_mesh(),
        out_type=[jax.ShapeDtypeStruct((NW, E_PAD), i32)] * nj
        + [jax.ShapeDtypeStruct((NW, 16), i32)] * nj,
        compiler_params=_SC_PARAMS,
        scratch_types=[
            pltpu.VMEM((DCH,), i32),
            pltpu.VMEM((E_PAD + 16,), i32),
            pltpu.VMEM((16,), i32),
        ],
    )
    def k(*refs):
        dst_hbms = refs[:nj]
        hits_hbms = refs[nj:2 * nj]
        cnt_hbms = refs[2 * nj:3 * nj]
        dbuf_v, hits_v, cnt_v = refs[3 * nj:]
        wid = lax.axis_index("s") * 2 + lax.axis_index("c")
        lo = wid * _ROWS
        hi = lo + _ROWS
        lane = lax.broadcasted_iota(i32, (16,), 0)

        for j in range(nj):
            def chunk_body(kk, cnt, dst_hbm=dst_hbms[j]):
                pltpu.sync_copy(dst_hbm.at[pl.ds(kk * DCH, DCH)], dbuf_v)

                def vbody(v, cnt):
                    d = dbuf_v[pl.ds(v * 16, 16)]
                    msk = (d >= lo) & (d < hi)
                    eid = kk * DCH + v * 16 + lane
                    packed = ((d - lo) << 16) | eid
                    pos = plsc.cumsum(msk.astype(i32))
                    plsc.store_scatter(hits_v, [cnt + pos - 1], packed, mask=msk)
                    return cnt + pos[15]

                return lax.fori_loop(0, DCH // 16, vbody, cnt, unroll=False)

            cnt = lax.fori_loop(0, nch, chunk_body, 0, unroll=False)

            cnt_v[...] = jnp.zeros((16,), i32) + cnt
            pltpu.sync_copy(cnt_v, cnt_hbms[j].at[wid])
            for c in range(E_PAD // _HCH):
                @pl.when(c * _HCH < cnt)
                def _(c=c, j=j):
                    pltpu.sync_copy(hits_v.at[pl.ds(c * _HCH, _HCH)],
                                    hits_hbms[j].at[wid, pl.ds(c * _HCH, _HCH)])

    outs = k(*dsts)
    return outs[:nj], outs[nj:]


def _sc_segmax3(ms, hits, cnts):
    G = 64
    nj = len(ms)

    @functools.partial(
        pl.kernel,
        mesh=_sc_mesh(),
        out_type=[jax.ShapeDtypeStruct((A_PAD, HH), f32)] * nj,
        compiler_params=_SC_PARAMS,
        scratch_types=[
            pltpu.VMEM((_ROWS, HH), f32),
            pltpu.VMEM((E_PAD + 16,), i32),
            pltpu.VMEM((16,), i32),
            pltpu.VMEM((G,), i32),
            pltpu.VMEM((G,), i32),
            pltpu.VMEM((G, HH), f32),
            pltpu.VMEM((G, HH), f32),
            pltpu.SemaphoreType.DMA,
            pltpu.SemaphoreType.DMA,
        ],
    )
    def k(*refs):
        m_hbms = refs[:nj]
        hits_hbms = refs[nj:2 * nj]
        cnt_hbms = refs[2 * nj:3 * nj]
        agg_hbms = refs[3 * nj:4 * nj]
        (acc_v, hits_v, cnt_v, idxa_v, idxb_v,
         rowsa_v, rowsb_v, sema, semb) = refs[4 * nj:]
        wid = lax.axis_index("s") * 2 + lax.axis_index("c")
        lo = wid * _ROWS
        lane = lax.broadcasted_iota(i32, (16,), 0)
        neginf = jnp.full((16,), -jnp.inf, f32)

        for j in range(nj):
            m_hbm = m_hbms[j]
            pltpu.sync_copy(cnt_hbms[j].at[wid], cnt_v)
            cnt = cnt_v[...][0]
            for c in range(E_PAD // _HCH):
                @pl.when(c * _HCH < cnt)
                def _(c=c, j=j):
                    pltpu.sync_copy(hits_hbms[j].at[wid, pl.ds(c * _HCH, _HCH)],
                                    hits_v.at[pl.ds(c * _HCH, _HCH)])

            def init_row(r, _):
                for c in range(HH // 16):
                    acc_v[r, pl.ds(c * 16, 16)] = neginf
                return 0

            lax.fori_loop(0, _ROWS, init_row, 0, unroll=False)

            ng = (cnt + G - 1) // G

            def stage_and_start(g, idxt_v, rows_v, sem):
                base = g * G
                for vv in range(G // 16):
                    pos = base + vv * 16
                    p = hits_v[pl.ds(pos, 16)]
                    valid = (pos + lane) < cnt
                    idxt_v[pl.ds(vv * 16, 16)] = jnp.where(valid, p & 0xFFFF, 0)
                return pltpu.async_copy(m_hbm.at[idxt_v], rows_v, sem)

            def accum(g, rows_v):
                base = g * G
                count = jnp.minimum(G, cnt - base)

                def rbody(r, _):
                    pk = hits_v[pl.ds(base + r, 16)][0]
                    off = pk >> 16

                    @pl.when(r < count)
                    def _():
                        for c in range(HH // 16):
                            sl = pl.ds(c * 16, 16)
                            acc_v[off, sl] = jnp.maximum(acc_v[off, sl],
                                                         rows_v[r, sl])

                    return 0

                lax.fori_loop(0, G, rbody, 0, unroll=False)

            @pl.when(ng > 0)
            def _():
                stage_and_start(0, idxa_v, rowsa_v, sema)

            npair = (ng + 1) // 2

            def pbody(p, _):
                e = 2 * p
                o = e + 1

                @pl.when(o < ng)
                def _():
                    stage_and_start(o, idxb_v, rowsb_v, semb)

                pltpu.make_async_copy(m_hbm.at[idxa_v], rowsa_v, sema).wait()
                accum(e, rowsa_v)

                @pl.when(o + 1 < ng)
                def _():
                    stage_and_start(o + 1, idxa_v, rowsa_v, sema)

                @pl.when(o < ng)
                def _():
                    pltpu.make_async_copy(m_hbm.at[idxb_v], rowsb_v, semb).wait()
                    accum(o, rowsb_v)

                return 0

            lax.fori_loop(0, npair, pbody, 0, unroll=False)

            pltpu.sync_copy(acc_v, agg_hbms[j].at[pl.ds(lo, _ROWS)])

    return k(*ms, *hits, *cnts)


def _mm(x, w):
    def body(x_ref, w_ref, o_ref):
        o_ref[...] = jnp.dot(x_ref[...], w_ref[...], preferred_element_type=f32)

    return pl.pallas_call(
        body,
        out_shape=jax.ShapeDtypeStruct((x.shape[0], w.shape[1]), f32),
    )(x, w)


def _edge_init(attr, w1, b1, w2, b2):
    R = 2048
    grid = E_PAD // R

    def body(a_ref, w1_ref, b1_ref, w2_ref, b2_ref, o_ref):
        h = jnp.maximum(
            jnp.dot(a_ref[...], w1_ref[...], preferred_element_type=f32)
            + b1_ref[...], 0.0)
        o_ref[...] = jnp.dot(h, w2_ref[...], preferred_element_type=f32) + b2_ref[...]

    full = lambda s: pl.BlockSpec(s, lambda i: (0, 0))
    return pl.pallas_call(
        body,
        grid=(grid,),
        in_specs=[
            pl.BlockSpec((R, 16), lambda i: (i, 0)),
            full((16, HH)), full((1, HH)), full((HH, HH)), full((1, HH)),
        ],
        out_specs=pl.BlockSpec((R, HH), lambda i: (i, 0)),
        out_shape=jax.ShapeDtypeStruct((E_PAD, HH), f32),
    )(attr, w1, b1.reshape(1, HH), w2, b2.reshape(1, HH))


def _edge_mlp(ea, src0, src1, w1e, b1, w2, b2, wf1, bf1, wf2, bf2):
    R = 2048
    grid = E_PAD // R
    pairs = list(src0) + list(src1)

    def body(ea_ref, *refs):
        gr = refs[:len(pairs)]
        refs = refs[len(pairs):]
        ur = []
        i = 0
        for g, u in pairs:
            if u is not None:
                ur.append(refs[i]); i += 1
            else:
                ur.append(None)
        (w1e_ref, b1_ref, w2_ref, b2_ref,
         wf1_ref, bf1_ref, wf2_ref, bf2_ref, ean_ref, m_ref) = refs[i:]

        def proj(j):
            return (jnp.dot(gr[j][...], ur[j][...], preferred_element_type=f32)
                    if ur[j] is not None else gr[j][...])

        a = ea_ref[...]
        pre = jnp.dot(a, w1e_ref[...], preferred_element_type=f32) + b1_ref[...]
        for j in range(len(pairs)):
            pre = pre + proj(j)
        h = jnp.maximum(pre, 0.0)
        ean = a + jnp.dot(h, w2_ref[...], preferred_element_type=f32) + b2_ref[...]
        ean_ref[...] = ean
        h2 = jnp.maximum(
            jnp.dot(ean, wf1_ref[...], preferred_element_type=f32) + bf1_ref[...], 0.0)
        m_ref[...] = jnp.dot(h2, wf2_ref[...], preferred_element_type=f32) + bf2_ref[...]

    eb = pl.BlockSpec((R, HH), lambda i: (i, 0))
    full = lambda s: pl.BlockSpec(s, lambda i: (0, 0))
    ins = [ea]
    specs = [eb]
    for g, u in pairs:
        ins.append(g)
        kg = g.shape[1]
        specs.append(pl.BlockSpec((R, kg), lambda i: (i, 0)))
    for g, u in pairs:
        if u is not None:
            ins.append(u)
            specs.append(full(u.shape))
    ins += [w1e, b1.reshape(1, HH), w2, b2.reshape(1, HH),
            wf1, bf1.reshape(1, HH), wf2, bf2.reshape(1, HH)]
    specs += [full((HH, HH)), full((1, HH)), full((HH, HH)), full((1, HH)),
              full((HH, HH)), full((1, HH)), full((HH, HH)), full((1, HH))]
    return pl.pallas_call(
        body,
        grid=(grid,),
        in_specs=specs,
        out_specs=[eb, eb],
        out_shape=[jax.ShapeDtypeStruct((E_PAD, HH), f32),
                   jax.ShapeDtypeStruct((E_PAD, HH), f32)],
    )(*ins)


def _agent_update(xa, a0, a1, a2):
    def body(x_ref, a0_ref, a1_ref, a2_ref, o_ref):
        def fix(v):
            return jnp.where(jnp.isfinite(v), v, 0.0)
        o_ref[...] = x_ref[...] + jnp.maximum(
            jnp.maximum(fix(a0_ref[...]), fix(a1_ref[...])), fix(a2_ref[...]))

    return pl.pallas_call(
        body,
        out_shape=jax.ShapeDtypeStruct((A_PAD, HH), f32),
    )(xa, a0, a1, a2)


def _field(xa, act, w1v, w1a, b1, w2, b2):
    def body(x_ref, a_ref, w1v_ref, w1a_ref, b1_ref, w2_ref, b2_ref, o_ref):
        h = jnp.maximum(
            jnp.dot(x_ref[...], w1v_ref[...], preferred_element_type=f32)
            + jnp.dot(a_ref[...], w1a_ref[...], preferred_element_type=f32)
            + b1_ref[...], 0.0)
        o_ref[...] = jnp.dot(h, w2_ref[...], preferred_element_type=f32) + b2_ref[...]

    return pl.pallas_call(
        body,
        out_shape=jax.ShapeDtypeStruct((A_PAD, 1), f32),
    )(xa, act, w1v, w1a, b1.reshape(1, HH), w2, b2.reshape(1, 1))


def kernel(x_obstacle, x_agent, x_goal, action,
           edge_index_oa, edge_attr_oa,
           edge_index_aa, edge_attr_aa,
           edge_index_ga, edge_attr_ga, params):
    types = ["oa", "aa", "ga"]
    ei = {"oa": edge_index_oa, "aa": edge_index_aa, "ga": edge_index_ga}
    eattr = {"oa": edge_attr_oa, "aa": edge_attr_aa, "ga": edge_attr_ga}

    xo_raw = _padr(x_obstacle, O_PAD)
    xa_raw = _padr(x_agent, A_PAD)
    xg_raw = _padr(x_goal, G_PAD)
    W_emb = params["W_embed"]
    xo = _mm(xo_raw, W_emb)
    xa = _mm(xa_raw, W_emb)
    xg = _mm(xg_raw, W_emb)
    act = _padr(action, A_PAD)

    ea, idx0, idx1 = {}, {}, {}
    for t in types:
        pp = params["ee_" + t]
        ea[t] = _edge_init(_padr(eattr[t], E_PAD), pp["W1"], pp["b1"], pp["W2"], pp["b2"])
        e = ei[t]
        idx0[t] = jnp.pad(e[0], (0, E_PAD - e.shape[1]), constant_values=0).astype(i32)
        idx1[t] = jnp.pad(e[1], (0, E_PAD - e.shape[1]), constant_values=0).astype(i32)

    dsts = [jnp.pad(ei[t][1], (0, E_PAD - ei[t].shape[1]),
                    constant_values=SENTINEL).astype(i32) for t in types]
    hits_l, cnts_l = _sc_hits3(dsts)

    r6 = _sc_gather_multi(
        [xo_raw, xa_raw, xg_raw, xa_raw, xa_raw, xa_raw],
        [idx0["oa"], idx0["aa"], idx0["ga"],
         idx1["oa"], idx1["aa"], idx1["ga"]],
        staged={0: [0], 1: [1, 3, 4, 5], 2: [2]})
    rx0 = dict(zip(types, r6[:3]))
    rx1 = dict(zip(types, r6[3:]))

    def w1split(l, t):
        W1 = params["em_%d_%s" % (l, t)]["W1"]
        return W1[:HH], W1[HH:2 * HH], W1[2 * HH:]

    for l in range(2):
        gathered = {}
        if l == 1:
            xa_lo, xa_hi = xa[:, :128], xa[:, 128:]
            idx_l1 = [idx0["aa"], idx1["oa"], idx1["aa"], idx1["ga"]]
            g8 = _sc_gather_multi(
                [xa_lo, xa_hi] * 4,
                [ix for ix in idx_l1 for _ in range(2)],
                staged={0: [0, 2, 4, 6], 1: [1, 3, 5, 7]})
            halves = {"aa_src": g8[0:2], "oa_dst": g8[2:4],
                      "aa_dst": g8[4:6], "ga_dst": g8[6:8]}
        ms = []
        for t in types:
            em = params["em_%d_%s" % (l, t)]
            W1e, W1s, W1d = w1split(l, t)
            if l == 0:
                src0 = [(rx0[t], _mm(W_emb, W1s))]
                src1 = [(rx1[t], _mm(W_emb, W1d))]
            else:
                if t == "aa":
                    glo, ghi = halves["aa_src"]
                    src0 = [(glo, W1s[:128]), (ghi, W1s[128:])]
                else:
                    src0 = [(rx0[t], _mm(W_emb, W1s))]
                glo, ghi = halves[t + "_dst"]
                src1 = [(glo, W1d[:128]), (ghi, W1d[128:])]
            fx = params["fx_%d_%s" % (l, t)]
            ea[t], m = _edge_mlp(ea[t], src0, src1,
                                 W1e, em["b1"], em["W2"], em["b2"],
                                 fx["W1"], fx["b1"], fx["W2"], fx["b2"])
            ms.append(m)
        aggs = [_sc_segmax3([ms[i]], [hits_l[i]], [cnts_l[i]])[0]
                for i in range(3)]
        xa = _agent_update(xa, aggs[0], aggs[1], aggs[2])

    fp = params["field"]
    W1f = fp["W1"]
    out = _field(xa, act, W1f[:HH], W1f[HH:], fp["b1"], fp["W2"], fp["b2"])
    return out[:x_agent.shape[0], 0]
